# Initial kernel scaffold; baseline (speedup 1.0000x reference)
import jax
import jax.numpy as jnp
from jax.experimental import pallas as pl


def kernel(x_n0, x_n1, he_index_n0, he_index_n1, max_he_id, k_W_n0, k_b_n0, q_W_n0, q_b_n0, a_W_n0, a_b_n0, skip_W_n0, skip_b_n0, ln_g_n0, ln_b_n0, k_W_n1, k_b_n1, q_W_n1, q_b_n1, a_W_n1, a_b_n1, skip_W_n1, skip_b_n1, ln_g_n1, ln_b_n1):
    raise NotImplementedError("write your pallas kernel here")


# speedup vs baseline: 57.2889x; 57.2889x over previous
import functools
import math

import jax
import jax.numpy as jnp
from jax import lax
from jax.experimental import pallas as pl
from jax.experimental.pallas import tpu as pltpu
from jax.experimental.pallas import tpu_sc as plsc

N = 10000
D = 128
OUT = 128
HEADS = 8
DK = OUT // HEADS
E = 160000
NC = 2
NS = 16
NW = NC * NS
CHUNK = 128
NCHUNK = E // CHUNK
SLAB = 640


def _per_sub_rows(sid, copy_fn):
    @pl.when(sid < NS - 1)
    def _():
        copy_fn(pl.multiple_of(sid * SLAB, SLAB), SLAB)

    @pl.when(sid == NS - 1)
    def _():
        copy_fn((NS - 1) * SLAB, N - (NS - 1) * SLAB)
@functools.cache
def _mesh():
    return plsc.VectorSubcoreMesh(core_axis_name="c", subcore_axis_name="s",
                                  num_cores=NC, num_subcores=NS)

f32 = jnp.float32


def _pre_body(x0, x1, w0, b0, w1, b1, hk0, q0, s0, hk1, q1, s1):
    r0 = jnp.dot(x0[...], w0[...], preferred_element_type=f32) + b0[...]
    hk0[...] = r0[:, 0:OUT]
    q0[...] = r0[:, OUT:2 * OUT]
    s0[...] = r0[:, 2 * OUT:3 * OUT]
    r1 = jnp.dot(x1[...], w1[...], preferred_element_type=f32) + b1[...]
    hk1[...] = r1[:, 0:OUT]
    q1[...] = r1[:, OUT:2 * OUT]
    s1[...] = r1[:, 2 * OUT:3 * OUT]


def _dense_pre(x0, x1, wcat0, bcat0, wcat1, bcat1):
    blk = 2000
    grid = N // blk
    io = pl.BlockSpec((blk, D), lambda i: (i, 0))
    w = pl.BlockSpec((D, 3 * OUT), lambda i: (0, 0))
    b = pl.BlockSpec((1, 3 * OUT), lambda i: (0, 0))
    o = pl.BlockSpec((blk, OUT), lambda i: (i, 0))
    sh = jax.ShapeDtypeStruct((N, OUT), f32)
    return pl.pallas_call(
        _pre_body, grid=(grid,),
        in_specs=[io, io, w, b, w, b],
        out_specs=[o] * 6,
        out_shape=[sh] * 6,
    )(x0, x1, wcat0, bcat0, wcat1, bcat1)


def _sc_hef_body(hk0, hk1, ni0, hi0, ni1, hi1, zeros, out,
                 nv, hv, rows, acc, sem):
    cid = lax.axis_index("c")
    sid = lax.axis_index("s")
    wid = sid * NC + cid
    _per_sub_rows(sid, lambda off, sz: pltpu.sync_copy(
        zeros.at[pl.ds(off, sz)], acc.at[pl.ds(off, sz)]))
    plsc.subcore_barrier()
    for hk, ni, hi in ((hk0, ni0, hi0), (hk1, ni1, hi1)):
        @pl.loop(0, (NCHUNK + NW - 1) // NW)
        def _(i):
            c = wid + i * NW

            @pl.when(c < NCHUNK)
            def _():
                base = pl.multiple_of(c * CHUNK, CHUNK)
                pltpu.sync_copy(ni.at[pl.ds(base, CHUNK)], nv)
                pltpu.sync_copy(hi.at[pl.ds(base, CHUNK)], hv)
                pltpu.async_copy(hk.at[nv], rows, sem).wait()
                pltpu.sync_copy(rows, acc.at[hv], add=True)
    plsc.subcore_barrier()
    _per_sub_rows(sid, lambda off, sz: pltpu.sync_copy(
        acc.at[pl.ds(off, sz)], out.at[cid, pl.ds(off, sz)]))


def _sc_hef(hk0, hk1, ni0, hi0, ni1, hi1, zeros):
    return pl.kernel(
        _sc_hef_body,
        out_type=jax.ShapeDtypeStruct((NC, N, OUT), f32),
        mesh=_mesh(),
        scratch_types=[
            pltpu.VMEM((CHUNK,), jnp.int32),
            pltpu.VMEM((CHUNK,), jnp.int32),
            pltpu.VMEM((CHUNK, OUT), f32),
            pltpu.VMEM_SHARED((N, OUT), f32),
            pltpu.SemaphoreType.DMA,
        ],
    )(hk0, hk1, ni0, hi0, ni1, hi1, zeros)


def _add2_body(p, o):
    o[...] = p[0] + p[1]


def _combine(parts):
    blk = 2000
    return pl.pallas_call(
        _add2_body, grid=(N // blk,),
        in_specs=[pl.BlockSpec((NC, blk, OUT), lambda i: (0, i, 0))],
        out_specs=pl.BlockSpec((blk, OUT), lambda i: (i, 0)),
        out_shape=jax.ShapeDtypeStruct((N, OUT), f32),
    )(parts)


def _sc_qk_body(q0, q1, hef, ni0, hi0, ni1, hi1,
                qg0, kg0, qg1, kg1, nv, hv, qrows, krows, sem, sem2):
    cid = lax.axis_index("c")
    sid = lax.axis_index("s")
    wid = sid * NC + cid
    for q, ni, hi, qg, kg in ((q0, ni0, hi0, qg0, kg0),
                              (q1, ni1, hi1, qg1, kg1)):
        @pl.loop(0, (NCHUNK + NW - 1) // NW)
        def _(i):
            c = wid + i * NW

            @pl.when(c < NCHUNK)
            def _():
                base = pl.multiple_of(c * CHUNK, CHUNK)
                pltpu.sync_copy(ni.at[pl.ds(base, CHUNK)], nv)
                pltpu.sync_copy(hi.at[pl.ds(base, CHUNK)], hv)
                cp1 = pltpu.async_copy(q.at[nv], qrows, sem)
                cp2 = pltpu.async_copy(hef.at[hv], krows, sem2)
                cp1.wait()
                cp2.wait()
                pltpu.sync_copy(qrows, qg.at[pl.ds(base, CHUNK)])
                pltpu.sync_copy(krows, kg.at[pl.ds(base, CHUNK)])


def _sc_qk(q0, q1, hef, ni0, hi0, ni1, hi1):
    sh = jax.ShapeDtypeStruct((E, OUT), f32)
    return pl.kernel(
        _sc_qk_body,
        out_type=(sh, sh, sh, sh),
        mesh=_mesh(),
        scratch_types=[
            pltpu.VMEM((CHUNK,), jnp.int32),
            pltpu.VMEM((CHUNK,), jnp.int32),
            pltpu.VMEM((CHUNK, OUT), f32),
            pltpu.VMEM((CHUNK, OUT), f32),
            pltpu.SemaphoreType.DMA,
            pltpu.SemaphoreType.DMA,
        ],
    )(q0, q1, hef, ni0, hi0, ni1, hi1)


def _alpha_body(qg, kg, g, alpha, mx):
    i = pl.program_id(0)
    a = jnp.dot(qg[...] * kg[...], g[...],
                preferred_element_type=f32) * (1.0 / math.sqrt(DK))
    alpha[...] = a
    bm = jnp.max(a, axis=0, keepdims=True)
    prev = jnp.where(i == 0, jnp.full((1, HEADS), -jnp.inf, f32), mx[...])
    mx[...] = jnp.maximum(prev, bm)


def _alpha_max(qg, kg, g):
    blk = 2000
    io = pl.BlockSpec((blk, OUT), lambda i: (i, 0))
    return pl.pallas_call(
        _alpha_body, grid=(E // blk,),
        in_specs=[io, io, pl.BlockSpec((OUT, HEADS), lambda i: (0, 0))],
        out_specs=[pl.BlockSpec((blk, HEADS), lambda i: (i, 0)),
                   pl.BlockSpec((1, HEADS), lambda i: (0, 0))],
        out_shape=[jax.ShapeDtypeStruct((E, HEADS), f32),
                   jax.ShapeDtypeStruct((1, HEADS), f32)],
    )(qg, kg, g)


def _val_body(alpha, mx, kg, gt, val, se):
    i = pl.program_id(0)
    w = jnp.exp(alpha[...] - mx[...])
    w128 = jnp.dot(w, gt[...], preferred_element_type=f32)
    val[...] = kg[...] * w128
    prev = jnp.where(i == 0, jnp.zeros((1, OUT), f32), se[...])
    se[...] = prev + jnp.sum(w128, axis=0, keepdims=True)


def _val(alpha, mx, kg, gt):
    blk = 2000
    return pl.pallas_call(
        _val_body, grid=(E // blk,),
        in_specs=[pl.BlockSpec((blk, HEADS), lambda i: (i, 0)),
                  pl.BlockSpec((1, HEADS), lambda i: (0, 0)),
                  pl.BlockSpec((blk, OUT), lambda i: (i, 0)),
                  pl.BlockSpec((HEADS, OUT), lambda i: (0, 0))],
        out_specs=[pl.BlockSpec((blk, OUT), lambda i: (i, 0)),
                   pl.BlockSpec((1, OUT), lambda i: (0, 0))],
        out_shape=[jax.ShapeDtypeStruct((E, OUT), f32),
                   jax.ShapeDtypeStruct((1, OUT), f32)],
    )(alpha, mx, kg, gt)


def _sc_nodeout_body(val0, val1, ni0, ni1, zeros, out,
                     nv, rows, acc, sem):
    cid = lax.axis_index("c")
    sid = lax.axis_index("s")
    wid = sid * NC + cid
    for t, (val, ni) in enumerate(((val0, ni0), (val1, ni1))):
        _per_sub_rows(sid, lambda off, sz: pltpu.sync_copy(
            zeros.at[pl.ds(off, sz)], acc.at[pl.ds(off, sz)]))
        plsc.subcore_barrier()

        @pl.loop(0, (NCHUNK + NW - 1) // NW)
        def _(i):
            c = wid + i * NW

            @pl.when(c < NCHUNK)
            def _():
                base = pl.multiple_of(c * CHUNK, CHUNK)
                pltpu.sync_copy(ni.at[pl.ds(base, CHUNK)], nv)
                pltpu.async_copy(val.at[pl.ds(base, CHUNK)], rows, sem).wait()
                pltpu.sync_copy(rows, acc.at[nv], add=True)
        plsc.subcore_barrier()
        _per_sub_rows(sid, lambda off, sz: pltpu.sync_copy(
            acc.at[pl.ds(off, sz)], out.at[t, cid, pl.ds(off, sz)]))
        plsc.subcore_barrier()


def _sc_nodeout(val0, val1, ni0, ni1, zeros):
    return pl.kernel(
        _sc_nodeout_body,
        out_type=jax.ShapeDtypeStruct((2, NC, N, OUT), f32),
        mesh=_mesh(),
        scratch_types=[
            pltpu.VMEM((CHUNK,), jnp.int32),
            pltpu.VMEM((CHUNK, OUT), f32),
            pltpu.VMEM_SHARED((N, OUT), f32),
            pltpu.SemaphoreType.DMA,
        ],
    )(val0, val1, ni0, ni1, zeros)


def _post_body(parts, se, skip, aw, ab, g, b, o):
    rec = 1.0 / se[...]
    no = (parts[0] + parts[1]) * rec
    merged = jnp.dot(no, aw[...], preferred_element_type=f32) + ab[...]
    y = merged + skip[...]
    mu = jnp.mean(y, axis=-1, keepdims=True)
    var = jnp.mean((y - mu) ** 2, axis=-1, keepdims=True)
    o[...] = (y - mu) * lax.rsqrt(var + 1e-5) * g[...] + b[...]


def _post(parts, se, skip, aw, ab, g, b):
    blk = 2000
    return pl.pallas_call(
        _post_body, grid=(N // blk,),
        in_specs=[pl.BlockSpec((NC, blk, OUT), lambda i: (0, i, 0)),
                  pl.BlockSpec((1, OUT), lambda i: (0, 0)),
                  pl.BlockSpec((blk, OUT), lambda i: (i, 0)),
                  pl.BlockSpec((OUT, OUT), lambda i: (0, 0)),
                  pl.BlockSpec((1, OUT), lambda i: (0, 0)),
                  pl.BlockSpec((1, OUT), lambda i: (0, 0)),
                  pl.BlockSpec((1, OUT), lambda i: (0, 0))],
        out_specs=pl.BlockSpec((blk, OUT), lambda i: (i, 0)),
        out_shape=jax.ShapeDtypeStruct((N, OUT), f32),
    )(parts, se, skip, aw, ab, g, b)


def kernel(x_n0, x_n1, he_index_n0, he_index_n1, max_he_id,
           k_W_n0, k_b_n0, q_W_n0, q_b_n0, a_W_n0, a_b_n0,
           skip_W_n0, skip_b_n0, ln_g_n0, ln_b_n0,
           k_W_n1, k_b_n1, q_W_n1, q_b_n1, a_W_n1, a_b_n1,
           skip_W_n1, skip_b_n1, ln_g_n1, ln_b_n1):
    del max_he_id

    ni0, hi0 = he_index_n0[0], he_index_n0[1]
    ni1, hi1 = he_index_n1[0], he_index_n1[1]

    wcat0 = jnp.concatenate([k_W_n0, q_W_n0, skip_W_n0], axis=1)
    bcat0 = jnp.concatenate([k_b_n0, q_b_n0, skip_b_n0])[None, :]
    wcat1 = jnp.concatenate([k_W_n1, q_W_n1, skip_W_n1], axis=1)
    bcat1 = jnp.concatenate([k_b_n1, q_b_n1, skip_b_n1])[None, :]

    eye = jnp.eye(HEADS, dtype=f32)
    g = jnp.repeat(eye, DK, axis=0)
    gt = jnp.repeat(eye, DK, axis=1)
    zeros = jnp.zeros((N, OUT), f32)

    hk0, q0, s0, hk1, q1, s1 = _dense_pre(x_n0, x_n1, wcat0, bcat0, wcat1, bcat1)

    hef_parts = _sc_hef(hk0, hk1, ni0, hi0, ni1, hi1, zeros)
    hef = _combine(hef_parts)

    qg0, kg0, qg1, kg1 = _sc_qk(q0, q1, hef, ni0, hi0, ni1, hi1)

    alpha0, mx0 = _alpha_max(qg0, kg0, g)
    alpha1, mx1 = _alpha_max(qg1, kg1, g)
    val0, se0 = _val(alpha0, mx0, kg0, gt)
    val1, se1 = _val(alpha1, mx1, kg1, gt)

    parts = _sc_nodeout(val0, val1, ni0, ni1, zeros)

    out0 = _post(parts[0], se0, s0, a_W_n0, a_b_n0[None, :],
                 ln_g_n0[None, :], ln_b_n0[None, :])
    out1 = _post(parts[1], se1, s1, a_W_n1, a_b_n1[None, :],
                 ln_g_n1[None, :], ln_b_n1[None, :])
    return (out0, out1)


# speedup vs baseline: 79.0728x; 1.3802x over previous
import functools
import math

import jax
import jax.numpy as jnp
from jax import lax
from jax.experimental import pallas as pl
from jax.experimental.pallas import tpu as pltpu
from jax.experimental.pallas import tpu_sc as plsc

N = 10000
D = 128
OUT = 128
HEADS = 8
DK = OUT // HEADS
E = 160000
NC = 2
NS = 16
NW = NC * NS
CHUNK = 128
NCHUNK = E // CHUNK
SLAB = 640


def _per_sub_rows(sid, copy_fn):
    @pl.when(sid < NS - 1)
    def _():
        copy_fn(pl.multiple_of(sid * SLAB, SLAB), SLAB)

    @pl.when(sid == NS - 1)
    def _():
        copy_fn((NS - 1) * SLAB, N - (NS - 1) * SLAB)
@functools.cache
def _mesh():
    return plsc.VectorSubcoreMesh(core_axis_name="c", subcore_axis_name="s",
                                  num_cores=NC, num_subcores=NS)

f32 = jnp.float32


def _pre_body(x0, x1, w0, b0, w1, b1, hk0, q0, s0, hk1, q1, s1):
    r0 = jnp.dot(x0[...], w0[...], preferred_element_type=f32) + b0[...]
    hk0[...] = r0[:, 0:OUT]
    q0[...] = r0[:, OUT:2 * OUT]
    s0[...] = r0[:, 2 * OUT:3 * OUT]
    r1 = jnp.dot(x1[...], w1[...], preferred_element_type=f32) + b1[...]
    hk1[...] = r1[:, 0:OUT]
    q1[...] = r1[:, OUT:2 * OUT]
    s1[...] = r1[:, 2 * OUT:3 * OUT]


def _dense_pre(x0, x1, wcat0, bcat0, wcat1, bcat1):
    blk = 2000
    grid = N // blk
    io = pl.BlockSpec((blk, D), lambda i: (i, 0))
    w = pl.BlockSpec((D, 3 * OUT), lambda i: (0, 0))
    b = pl.BlockSpec((1, 3 * OUT), lambda i: (0, 0))
    o = pl.BlockSpec((blk, OUT), lambda i: (i, 0))
    sh = jax.ShapeDtypeStruct((N, OUT), f32)
    return pl.pallas_call(
        _pre_body, grid=(grid,),
        in_specs=[io, io, w, b, w, b],
        out_specs=[o] * 6,
        out_shape=[sh] * 6,
    )(x0, x1, wcat0, bcat0, wcat1, bcat1)


EPW = 4992
NFULL = EPW // CHUNK
TAIL = NFULL * CHUNK * NW


def _sc_hef_body(hk0, hk1, ni0, hi0, ni1, hi1, zeros, out,
                 nvf, nve, hv0, hv1, rows0, rows1, acc,
                 sg0, sg1, sh0, sh1, ss):
    cid = lax.axis_index("c")
    sid = lax.axis_index("s")
    wid = sid * NC + cid
    ebase = pl.multiple_of(wid * EPW, CHUNK)
    nch = jnp.where(wid < 2, NFULL + 1, NFULL)
    _per_sub_rows(sid, lambda off, sz: pltpu.sync_copy(
        zeros.at[pl.ds(off, sz)], acc.at[pl.ds(off, sz)]))
    plsc.subcore_barrier()
    for hk, ni, hi in ((hk0, ni0, hi0), (hk1, ni1, hi1)):
        pltpu.async_copy(ni.at[pl.ds(ebase, EPW)], nvf, ss).wait()

        @pl.when(wid < 2)
        def _():
            pltpu.sync_copy(ni.at[pl.ds(pl.multiple_of(TAIL + wid * CHUNK,
                                                       CHUNK), CHUNK)], nve)

        def issue(c, hv, rows, sg, sh):
            @pl.when(c < NFULL)
            def _():
                b = pl.multiple_of(c * CHUNK, CHUNK)
                pltpu.async_copy(hi.at[pl.ds(pl.multiple_of(ebase + b, CHUNK),
                                             CHUNK)], hv, sh)
                pltpu.async_copy(hk.at[nvf.at[pl.ds(b, CHUNK)]], rows, sg)

            @pl.when(c == NFULL)
            def _():
                off = pl.multiple_of(TAIL + wid * CHUNK, CHUNK)
                pltpu.async_copy(hi.at[pl.ds(off, CHUNK)], hv, sh)
                pltpu.async_copy(hk.at[nve], rows, sg)

        def finish(c, hv, rows, sg, sh):
            pltpu.make_async_copy(hi.at[pl.ds(0, CHUNK)], hv, sh).wait()
            pltpu.make_async_copy(hk.at[nve], rows, sg).wait()
            pltpu.sync_copy(rows, acc.at[hv], add=True)

        issue(0, hv0, rows0, sg0, sh0)

        @pl.loop(0, (NFULL + 2) // 2)
        def _(j):
            a = 2 * j
            b2 = 2 * j + 1

            @pl.when(a < nch)
            def _():
                @pl.when(b2 < nch)
                def _():
                    issue(b2, hv1, rows1, sg1, sh1)
                finish(a, hv0, rows0, sg0, sh0)

            @pl.when(b2 < nch)
            def _():
                @pl.when(b2 + 1 < nch)
                def _():
                    issue(b2 + 1, hv0, rows0, sg0, sh0)
                finish(b2, hv1, rows1, sg1, sh1)
    plsc.subcore_barrier()
    _per_sub_rows(sid, lambda off, sz: pltpu.sync_copy(
        acc.at[pl.ds(off, sz)], out.at[cid, pl.ds(off, sz)]))


def _sc_hef(hk0, hk1, ni0, hi0, ni1, hi1, zeros):
    return pl.kernel(
        _sc_hef_body,
        out_type=jax.ShapeDtypeStruct((NC, N, OUT), f32),
        mesh=_mesh(),
        scratch_types=[
            pltpu.VMEM((EPW,), jnp.int32),
            pltpu.VMEM((CHUNK,), jnp.int32),
            pltpu.VMEM((CHUNK,), jnp.int32),
            pltpu.VMEM((CHUNK,), jnp.int32),
            pltpu.VMEM((CHUNK, OUT), f32),
            pltpu.VMEM((CHUNK, OUT), f32),
            pltpu.VMEM_SHARED((N, OUT), f32),
            pltpu.SemaphoreType.DMA,
            pltpu.SemaphoreType.DMA,
            pltpu.SemaphoreType.DMA,
            pltpu.SemaphoreType.DMA,
            pltpu.SemaphoreType.DMA,
        ],
    )(hk0, hk1, ni0, hi0, ni1, hi1, zeros)


def _add2_body(p, o):
    o[...] = p[0] + p[1]


def _combine(parts):
    blk = 2000
    return pl.pallas_call(
        _add2_body, grid=(N // blk,),
        in_specs=[pl.BlockSpec((NC, blk, OUT), lambda i: (0, i, 0))],
        out_specs=pl.BlockSpec((blk, OUT), lambda i: (i, 0)),
        out_shape=jax.ShapeDtypeStruct((N, OUT), f32),
    )(parts)


def _sc_qk_body(q0, q1, hef, ni0, hi0, ni1, hi1,
                qg0, kg0, qg1, kg1,
                nvf, hvf, nve, hve, qr0, qr1, kr0, kr1,
                sq0, sq1, sk0, sk1, wq0, wq1, wk0, wk1, ss):
    cid = lax.axis_index("c")
    sid = lax.axis_index("s")
    wid = sid * NC + cid
    ebase = pl.multiple_of(wid * EPW, CHUNK)
    nch = jnp.where(wid < 2, NFULL + 1, NFULL)
    for q, ni, hi, qg, kg in ((q0, ni0, hi0, qg0, kg0),
                              (q1, ni1, hi1, qg1, kg1)):
        cpa = pltpu.async_copy(ni.at[pl.ds(ebase, EPW)], nvf, ss)
        cpb = pltpu.async_copy(hi.at[pl.ds(ebase, EPW)], hvf, ss)
        cpa.wait()
        cpb.wait()

        @pl.when(wid < 2)
        def _():
            toff = pl.multiple_of(TAIL + wid * CHUNK, CHUNK)
            pltpu.sync_copy(ni.at[pl.ds(toff, CHUNK)], nve)
            pltpu.sync_copy(hi.at[pl.ds(toff, CHUNK)], hve)

        def pre(c, qr, kr, sq, sk, wq, wk):
            @pl.when(c < nch)
            def _():
                @pl.when(c >= 2)
                def _():
                    pltpu.make_async_copy(qr, qg.at[pl.ds(0, CHUNK)], wq).wait()
                    pltpu.make_async_copy(kr, kg.at[pl.ds(0, CHUNK)], wk).wait()

                @pl.when(c < NFULL)
                def _():
                    b = pl.multiple_of(c * CHUNK, CHUNK)
                    pltpu.async_copy(q.at[nvf.at[pl.ds(b, CHUNK)]], qr, sq)
                    pltpu.async_copy(hef.at[hvf.at[pl.ds(b, CHUNK)]], kr, sk)

                @pl.when(c == NFULL)
                def _():
                    pltpu.async_copy(q.at[nve], qr, sq)
                    pltpu.async_copy(hef.at[hve], kr, sk)

        def proc(c, qr, kr, sq, sk, wq, wk):
            @pl.when(c < nch)
            def _():
                pltpu.make_async_copy(q.at[nve], qr, sq).wait()
                pltpu.make_async_copy(hef.at[hve], kr, sk).wait()
                woff = pl.multiple_of(
                    jnp.where(c < NFULL, ebase + c * CHUNK,
                              TAIL + wid * CHUNK), CHUNK)
                pltpu.async_copy(qr, qg.at[pl.ds(woff, CHUNK)], wq)
                pltpu.async_copy(kr, kg.at[pl.ds(woff, CHUNK)], wk)

        pre(0, qr0, kr0, sq0, sk0, wq0, wk0)
        pre(1, qr1, kr1, sq1, sk1, wq1, wk1)

        @pl.loop(0, (NFULL + 2) // 2)
        def _(j):
            a = 2 * j
            b2 = 2 * j + 1
            proc(a, qr0, kr0, sq0, sk0, wq0, wk0)
            pre(a + 2, qr0, kr0, sq0, sk0, wq0, wk0)
            proc(b2, qr1, kr1, sq1, sk1, wq1, wk1)
            pre(b2 + 2, qr1, kr1, sq1, sk1, wq1, wk1)

        pltpu.make_async_copy(qr0, qg.at[pl.ds(0, CHUNK)], wq0).wait()
        pltpu.make_async_copy(kr0, kg.at[pl.ds(0, CHUNK)], wk0).wait()
        pltpu.make_async_copy(qr1, qg.at[pl.ds(0, CHUNK)], wq1).wait()
        pltpu.make_async_copy(kr1, kg.at[pl.ds(0, CHUNK)], wk1).wait()


def _sc_qk(q0, q1, hef, ni0, hi0, ni1, hi1):
    sh = jax.ShapeDtypeStruct((E, OUT), f32)
    return pl.kernel(
        _sc_qk_body,
        out_type=(sh, sh, sh, sh),
        mesh=_mesh(),
        scratch_types=[
            pltpu.VMEM((EPW,), jnp.int32),
            pltpu.VMEM((EPW,), jnp.int32),
            pltpu.VMEM((CHUNK,), jnp.int32),
            pltpu.VMEM((CHUNK,), jnp.int32),
            pltpu.VMEM((CHUNK, OUT), f32),
            pltpu.VMEM((CHUNK, OUT), f32),
            pltpu.VMEM((CHUNK, OUT), f32),
            pltpu.VMEM((CHUNK, OUT), f32),
            pltpu.SemaphoreType.DMA,
            pltpu.SemaphoreType.DMA,
            pltpu.SemaphoreType.DMA,
            pltpu.SemaphoreType.DMA,
            pltpu.SemaphoreType.DMA,
            pltpu.SemaphoreType.DMA,
            pltpu.SemaphoreType.DMA,
            pltpu.SemaphoreType.DMA,
            pltpu.SemaphoreType.DMA,
        ],
    )(q0, q1, hef, ni0, hi0, ni1, hi1)


def _alpha_body(qg, kg, g, alpha, mx):
    i = pl.program_id(0)
    a = jnp.dot(qg[...] * kg[...], g[...],
                preferred_element_type=f32) * (1.0 / math.sqrt(DK))
    alpha[...] = a
    bm = jnp.max(a, axis=0, keepdims=True)
    prev = jnp.where(i == 0, jnp.full((1, HEADS), -jnp.inf, f32), mx[...])
    mx[...] = jnp.maximum(prev, bm)


def _alpha_max(qg, kg, g):
    blk = 2000
    io = pl.BlockSpec((blk, OUT), lambda i: (i, 0))
    return pl.pallas_call(
        _alpha_body, grid=(E // blk,),
        in_specs=[io, io, pl.BlockSpec((OUT, HEADS), lambda i: (0, 0))],
        out_specs=[pl.BlockSpec((blk, HEADS), lambda i: (i, 0)),
                   pl.BlockSpec((1, HEADS), lambda i: (0, 0))],
        out_shape=[jax.ShapeDtypeStruct((E, HEADS), f32),
                   jax.ShapeDtypeStruct((1, HEADS), f32)],
    )(qg, kg, g)


def _val_body(alpha, mx, kg, gt, val, se):
    i = pl.program_id(0)
    w = jnp.exp(alpha[...] - mx[...])
    w128 = jnp.dot(w, gt[...], preferred_element_type=f32)
    val[...] = kg[...] * w128
    prev = jnp.where(i == 0, jnp.zeros((1, OUT), f32), se[...])
    se[...] = prev + jnp.sum(w128, axis=0, keepdims=True)


def _val(alpha, mx, kg, gt):
    blk = 2000
    return pl.pallas_call(
        _val_body, grid=(E // blk,),
        in_specs=[pl.BlockSpec((blk, HEADS), lambda i: (i, 0)),
                  pl.BlockSpec((1, HEADS), lambda i: (0, 0)),
                  pl.BlockSpec((blk, OUT), lambda i: (i, 0)),
                  pl.BlockSpec((HEADS, OUT), lambda i: (0, 0))],
        out_specs=[pl.BlockSpec((blk, OUT), lambda i: (i, 0)),
                   pl.BlockSpec((1, OUT), lambda i: (0, 0))],
        out_shape=[jax.ShapeDtypeStruct((E, OUT), f32),
                   jax.ShapeDtypeStruct((1, OUT), f32)],
    )(alpha, mx, kg, gt)


def _sc_nodeout_body(val0, val1, ni0, ni1, zeros, out,
                     nv0, nv1, rv0, rv1, acc,
                     sn0, sn1, sv0, sv1):
    cid = lax.axis_index("c")
    sid = lax.axis_index("s")
    wid = sid * NC + cid
    ebase = pl.multiple_of(wid * EPW, CHUNK)
    nch = jnp.where(wid < 2, NFULL + 1, NFULL)
    for t, (val, ni) in enumerate(((val0, ni0), (val1, ni1))):
        _per_sub_rows(sid, lambda off, sz: pltpu.sync_copy(
            zeros.at[pl.ds(off, sz)], acc.at[pl.ds(off, sz)]))
        plsc.subcore_barrier()

        def issue(c, nv, rv, sn, sv):
            off = pl.multiple_of(
                jnp.where(c < NFULL, ebase + c * CHUNK,
                          TAIL + wid * CHUNK), CHUNK)
            pltpu.async_copy(ni.at[pl.ds(off, CHUNK)], nv, sn)
            pltpu.async_copy(val.at[pl.ds(off, CHUNK)], rv, sv)

        def finish(c, nv, rv, sn, sv):
            pltpu.make_async_copy(ni.at[pl.ds(0, CHUNK)], nv, sn).wait()
            pltpu.make_async_copy(val.at[pl.ds(0, CHUNK)], rv, sv).wait()
            pltpu.sync_copy(rv, acc.at[nv], add=True)

        issue(0, nv0, rv0, sn0, sv0)

        @pl.loop(0, (NFULL + 2) // 2)
        def _(j):
            a = 2 * j
            b2 = 2 * j + 1

            @pl.when(a < nch)
            def _():
                @pl.when(b2 < nch)
                def _():
                    issue(b2, nv1, rv1, sn1, sv1)
                finish(a, nv0, rv0, sn0, sv0)

            @pl.when(b2 < nch)
            def _():
                @pl.when(b2 + 1 < nch)
                def _():
                    issue(b2 + 1, nv0, rv0, sn0, sv0)
                finish(b2, nv1, rv1, sn1, sv1)
        plsc.subcore_barrier()
        _per_sub_rows(sid, lambda off, sz: pltpu.sync_copy(
            acc.at[pl.ds(off, sz)], out.at[t, cid, pl.ds(off, sz)]))
        plsc.subcore_barrier()


def _sc_nodeout(val0, val1, ni0, ni1, zeros):
    return pl.kernel(
        _sc_nodeout_body,
        out_type=jax.ShapeDtypeStruct((2, NC, N, OUT), f32),
        mesh=_mesh(),
        scratch_types=[
            pltpu.VMEM((CHUNK,), jnp.int32),
            pltpu.VMEM((CHUNK,), jnp.int32),
            pltpu.VMEM((CHUNK, OUT), f32),
            pltpu.VMEM((CHUNK, OUT), f32),
            pltpu.VMEM_SHARED((N, OUT), f32),
            pltpu.SemaphoreType.DMA,
            pltpu.SemaphoreType.DMA,
            pltpu.SemaphoreType.DMA,
            pltpu.SemaphoreType.DMA,
        ],
    )(val0, val1, ni0, ni1, zeros)


def _post_body(parts, se, skip, aw, ab, g, b, o):
    rec = 1.0 / se[...]
    no = (parts[0] + parts[1]) * rec
    merged = jnp.dot(no, aw[...], preferred_element_type=f32) + ab[...]
    y = merged + skip[...]
    mu = jnp.mean(y, axis=-1, keepdims=True)
    var = jnp.mean((y - mu) ** 2, axis=-1, keepdims=True)
    o[...] = (y - mu) * lax.rsqrt(var + 1e-5) * g[...] + b[...]


def _post(parts, se, skip, aw, ab, g, b):
    blk = 2000
    return pl.pallas_call(
        _post_body, grid=(N // blk,),
        in_specs=[pl.BlockSpec((NC, blk, OUT), lambda i: (0, i, 0)),
                  pl.BlockSpec((1, OUT), lambda i: (0, 0)),
                  pl.BlockSpec((blk, OUT), lambda i: (i, 0)),
                  pl.BlockSpec((OUT, OUT), lambda i: (0, 0)),
                  pl.BlockSpec((1, OUT), lambda i: (0, 0)),
                  pl.BlockSpec((1, OUT), lambda i: (0, 0)),
                  pl.BlockSpec((1, OUT), lambda i: (0, 0))],
        out_specs=pl.BlockSpec((blk, OUT), lambda i: (i, 0)),
        out_shape=jax.ShapeDtypeStruct((N, OUT), f32),
    )(parts, se, skip, aw, ab, g, b)


def kernel(x_n0, x_n1, he_index_n0, he_index_n1, max_he_id,
           k_W_n0, k_b_n0, q_W_n0, q_b_n0, a_W_n0, a_b_n0,
           skip_W_n0, skip_b_n0, ln_g_n0, ln_b_n0,
           k_W_n1, k_b_n1, q_W_n1, q_b_n1, a_W_n1, a_b_n1,
           skip_W_n1, skip_b_n1, ln_g_n1, ln_b_n1):
    del max_he_id

    ni0, hi0 = he_index_n0[0], he_index_n0[1]
    ni1, hi1 = he_index_n1[0], he_index_n1[1]

    wcat0 = jnp.concatenate([k_W_n0, q_W_n0, skip_W_n0], axis=1)
    bcat0 = jnp.concatenate([k_b_n0, q_b_n0, skip_b_n0])[None, :]
    wcat1 = jnp.concatenate([k_W_n1, q_W_n1, skip_W_n1], axis=1)
    bcat1 = jnp.concatenate([k_b_n1, q_b_n1, skip_b_n1])[None, :]

    eye = jnp.eye(HEADS, dtype=f32)
    g = jnp.repeat(eye, DK, axis=0)
    gt = jnp.repeat(eye, DK, axis=1)
    zeros = jnp.zeros((N, OUT), f32)

    hk0, q0, s0, hk1, q1, s1 = _dense_pre(x_n0, x_n1, wcat0, bcat0, wcat1, bcat1)

    hef_parts = _sc_hef(hk0, hk1, ni0, hi0, ni1, hi1, zeros)
    hef = _combine(hef_parts)

    qg0, kg0, qg1, kg1 = _sc_qk(q0, q1, hef, ni0, hi0, ni1, hi1)

    alpha0, mx0 = _alpha_max(qg0, kg0, g)
    alpha1, mx1 = _alpha_max(qg1, kg1, g)
    val0, se0 = _val(alpha0, mx0, kg0, gt)
    val1, se1 = _val(alpha1, mx1, kg1, gt)

    parts = _sc_nodeout(val0, val1, ni0, ni1, zeros)

    out0 = _post(parts[0], se0, s0, a_W_n0, a_b_n0[None, :],
                 ln_g_n0[None, :], ln_b_n0[None, :])
    out1 = _post(parts[1], se1, s1, a_W_n1, a_b_n1[None, :],
                 ln_g_n1[None, :], ln_b_n1[None, :])
    return (out0, out1)


# speedup vs baseline: 111.0707x; 1.4047x over previous
import functools
import math

import jax
import jax.numpy as jnp
from jax import lax
from jax.experimental import pallas as pl
from jax.experimental.pallas import tpu as pltpu
from jax.experimental.pallas import tpu_sc as plsc

N = 10000
D = 128
OUT = 128
HEADS = 8
DK = OUT // HEADS
E = 160000
NC = 2
NS = 16
NW = NC * NS
CHUNK = 128
NCHUNK = E // CHUNK
SLAB = 640


def _per_sub_rows(sid, copy_fn):
    @pl.when(sid < NS - 1)
    def _():
        copy_fn(pl.multiple_of(sid * SLAB, SLAB), SLAB)

    @pl.when(sid == NS - 1)
    def _():
        copy_fn((NS - 1) * SLAB, N - (NS - 1) * SLAB)
@functools.cache
def _mesh():
    return plsc.VectorSubcoreMesh(core_axis_name="c", subcore_axis_name="s",
                                  num_cores=NC, num_subcores=NS)

f32 = jnp.float32


def _pre_body(x0, x1, w0, b0, w1, b1, hk0, q0, s0, hk1, q1, s1):
    r0 = jnp.dot(x0[...], w0[...], preferred_element_type=f32) + b0[...]
    hk0[...] = r0[:, 0:OUT]
    q0[...] = r0[:, OUT:2 * OUT]
    s0[...] = r0[:, 2 * OUT:3 * OUT]
    r1 = jnp.dot(x1[...], w1[...], preferred_element_type=f32) + b1[...]
    hk1[...] = r1[:, 0:OUT]
    q1[...] = r1[:, OUT:2 * OUT]
    s1[...] = r1[:, 2 * OUT:3 * OUT]


def _dense_pre(x0, x1, wcat0, bcat0, wcat1, bcat1):
    blk = 2000
    grid = N // blk
    io = pl.BlockSpec((blk, D), lambda i: (i, 0))
    w = pl.BlockSpec((D, 3 * OUT), lambda i: (0, 0))
    b = pl.BlockSpec((1, 3 * OUT), lambda i: (0, 0))
    o = pl.BlockSpec((blk, OUT), lambda i: (i, 0))
    sh = jax.ShapeDtypeStruct((N, OUT), f32)
    return pl.pallas_call(
        _pre_body, grid=(grid,),
        in_specs=[io, io, w, b, w, b],
        out_specs=[o] * 6,
        out_shape=[sh] * 6,
    )(x0, x1, wcat0, bcat0, wcat1, bcat1)


EPW = 4992
NFULL = EPW // CHUNK
TAIL = NFULL * CHUNK * NW


def _sc_hef_body(hk0, hk1, ni0, hi0, ni1, hi1, zeros, out,
                 nvf, nve, hv0, hv1, rows0, rows1, acc,
                 sg0, sg1, sh0, sh1, ss):
    cid = lax.axis_index("c")
    sid = lax.axis_index("s")
    wid = sid * NC + cid
    ebase = pl.multiple_of(wid * EPW, CHUNK)
    nch = jnp.where(wid < 2, NFULL + 1, NFULL)
    _per_sub_rows(sid, lambda off, sz: pltpu.sync_copy(
        zeros.at[pl.ds(off, sz)], acc.at[pl.ds(off, sz)]))
    plsc.subcore_barrier()
    for hk, ni, hi in ((hk0, ni0, hi0), (hk1, ni1, hi1)):
        pltpu.async_copy(ni.at[pl.ds(ebase, EPW)], nvf, ss).wait()

        @pl.when(wid < 2)
        def _():
            pltpu.sync_copy(ni.at[pl.ds(pl.multiple_of(TAIL + wid * CHUNK,
                                                       CHUNK), CHUNK)], nve)

        def issue(c, hv, rows, sg, sh):
            @pl.when(c < NFULL)
            def _():
                b = pl.multiple_of(c * CHUNK, CHUNK)
                pltpu.async_copy(hi.at[pl.ds(pl.multiple_of(ebase + b, CHUNK),
                                             CHUNK)], hv, sh)
                pltpu.async_copy(hk.at[nvf.at[pl.ds(b, CHUNK)]], rows, sg)

            @pl.when(c == NFULL)
            def _():
                off = pl.multiple_of(TAIL + wid * CHUNK, CHUNK)
                pltpu.async_copy(hi.at[pl.ds(off, CHUNK)], hv, sh)
                pltpu.async_copy(hk.at[nve], rows, sg)

        def finish(c, hv, rows, sg, sh):
            pltpu.make_async_copy(hi.at[pl.ds(0, CHUNK)], hv, sh).wait()
            pltpu.make_async_copy(hk.at[nve], rows, sg).wait()
            pltpu.sync_copy(rows, acc.at[hv], add=True)

        issue(0, hv0, rows0, sg0, sh0)

        @pl.loop(0, (NFULL + 2) // 2)
        def _(j):
            a = 2 * j
            b2 = 2 * j + 1

            @pl.when(a < nch)
            def _():
                @pl.when(b2 < nch)
                def _():
                    issue(b2, hv1, rows1, sg1, sh1)
                finish(a, hv0, rows0, sg0, sh0)

            @pl.when(b2 < nch)
            def _():
                @pl.when(b2 + 1 < nch)
                def _():
                    issue(b2 + 1, hv0, rows0, sg0, sh0)
                finish(b2, hv1, rows1, sg1, sh1)
    plsc.subcore_barrier()
    _per_sub_rows(sid, lambda off, sz: pltpu.sync_copy(
        acc.at[pl.ds(off, sz)], out.at[cid, pl.ds(off, sz)]))


def _sc_hef(hk0, hk1, ni0, hi0, ni1, hi1, zeros):
    return pl.kernel(
        _sc_hef_body,
        out_type=jax.ShapeDtypeStruct((NC, N, OUT), f32),
        mesh=_mesh(),
        scratch_types=[
            pltpu.VMEM((EPW,), jnp.int32),
            pltpu.VMEM((CHUNK,), jnp.int32),
            pltpu.VMEM((CHUNK,), jnp.int32),
            pltpu.VMEM((CHUNK,), jnp.int32),
            pltpu.VMEM((CHUNK, OUT), f32),
            pltpu.VMEM((CHUNK, OUT), f32),
            pltpu.VMEM_SHARED((N, OUT), f32),
            pltpu.SemaphoreType.DMA,
            pltpu.SemaphoreType.DMA,
            pltpu.SemaphoreType.DMA,
            pltpu.SemaphoreType.DMA,
            pltpu.SemaphoreType.DMA,
        ],
    )(hk0, hk1, ni0, hi0, ni1, hi1, zeros)


def _combine_norms_body(p, q0, q1, g, hef, mq0, mq1, mh):
    i = pl.program_id(0)
    h = p[0] + p[1]
    hef[...] = h

    def _nmax(x):
        n = jnp.sqrt(jnp.dot(x * x, g[...], preferred_element_type=f32))
        return jnp.max(n, axis=0, keepdims=True)

    neg = jnp.full((1, HEADS), -jnp.inf, f32)
    for ref, x in ((mq0, q0[...]), (mq1, q1[...]), (mh, h)):
        prev = jnp.where(i == 0, neg, ref[...])
        ref[...] = jnp.maximum(prev, _nmax(x))


def _combine_norms(parts, q0, q1, g):
    blk = 2000
    io = pl.BlockSpec((blk, OUT), lambda i: (i, 0))
    m = pl.BlockSpec((1, HEADS), lambda i: (0, 0))
    msh = jax.ShapeDtypeStruct((1, HEADS), f32)
    return pl.pallas_call(
        _combine_norms_body, grid=(N // blk,),
        in_specs=[pl.BlockSpec((NC, blk, OUT), lambda i: (0, i, 0)), io, io,
                  pl.BlockSpec((OUT, HEADS), lambda i: (0, 0))],
        out_specs=[pl.BlockSpec((blk, OUT), lambda i: (i, 0)), m, m, m],
        out_shape=[jax.ShapeDtypeStruct((N, OUT), f32), msh, msh, msh],
    )(parts, q0, q1, g)


def _sc_qk_body(q, hef, ni, hi,
                qg, kg,
                nvf, hvf, nve, hve, qr0, qr1, kr0, kr1,
                sq0, sq1, sk0, sk1, wq0, wq1, wk0, wk1, ss):
    cid = lax.axis_index("c")
    sid = lax.axis_index("s")
    wid = sid * NC + cid
    ebase = pl.multiple_of(wid * EPW, CHUNK)
    nch = jnp.where(wid < 2, NFULL + 1, NFULL)
    cpa = pltpu.async_copy(ni.at[pl.ds(ebase, EPW)], nvf, ss)
    cpb = pltpu.async_copy(hi.at[pl.ds(ebase, EPW)], hvf, ss)
    cpa.wait()
    cpb.wait()

    @pl.when(wid < 2)
    def _():
        toff = pl.multiple_of(TAIL + wid * CHUNK, CHUNK)
        pltpu.sync_copy(ni.at[pl.ds(toff, CHUNK)], nve)
        pltpu.sync_copy(hi.at[pl.ds(toff, CHUNK)], hve)

    def pre(c, qr, kr, sq, sk, wq, wk):
        @pl.when(c < nch)
        def _():
            @pl.when(c >= 2)
            def _():
                pltpu.make_async_copy(qr, qg.at[pl.ds(0, CHUNK)], wq).wait()
                pltpu.make_async_copy(kr, kg.at[pl.ds(0, CHUNK)], wk).wait()

            @pl.when(c < NFULL)
            def _():
                b = pl.multiple_of(c * CHUNK, CHUNK)
                pltpu.async_copy(q.at[nvf.at[pl.ds(b, CHUNK)]], qr, sq)
                pltpu.async_copy(hef.at[hvf.at[pl.ds(b, CHUNK)]], kr, sk)

            @pl.when(c == NFULL)
            def _():
                pltpu.async_copy(q.at[nve], qr, sq)
                pltpu.async_copy(hef.at[hve], kr, sk)

    def proc(c, qr, kr, sq, sk, wq, wk):
        @pl.when(c < nch)
        def _():
            pltpu.make_async_copy(q.at[nve], qr, sq).wait()
            pltpu.make_async_copy(hef.at[hve], kr, sk).wait()
            woff = pl.multiple_of(
                jnp.where(c < NFULL, ebase + c * CHUNK,
                          TAIL + wid * CHUNK), CHUNK)
            pltpu.async_copy(qr, qg.at[pl.ds(woff, CHUNK)], wq)
            pltpu.async_copy(kr, kg.at[pl.ds(woff, CHUNK)], wk)

    pre(jnp.int32(0), qr0, kr0, sq0, sk0, wq0, wk0)
    pre(jnp.int32(1), qr1, kr1, sq1, sk1, wq1, wk1)

    @pl.loop(0, (NFULL + 2) // 2)
    def _(j):
        a = 2 * j
        b2 = 2 * j + 1
        proc(a, qr0, kr0, sq0, sk0, wq0, wk0)
        pre(a + 2, qr0, kr0, sq0, sk0, wq0, wk0)
        proc(b2, qr1, kr1, sq1, sk1, wq1, wk1)
        pre(b2 + 2, qr1, kr1, sq1, sk1, wq1, wk1)

    pltpu.make_async_copy(qr0, qg.at[pl.ds(0, CHUNK)], wq0).wait()
    pltpu.make_async_copy(kr0, kg.at[pl.ds(0, CHUNK)], wk0).wait()
    pltpu.make_async_copy(qr1, qg.at[pl.ds(0, CHUNK)], wq1).wait()
    pltpu.make_async_copy(kr1, kg.at[pl.ds(0, CHUNK)], wk1).wait()


def _sc_qk(q, hef, ni, hi):
    sh = jax.ShapeDtypeStruct((E, OUT), f32)
    return pl.kernel(
        _sc_qk_body,
        out_type=(sh, sh),
        mesh=_mesh(),
        scratch_types=[
            pltpu.VMEM((EPW,), jnp.int32),
            pltpu.VMEM((EPW,), jnp.int32),
            pltpu.VMEM((CHUNK,), jnp.int32),
            pltpu.VMEM((CHUNK,), jnp.int32),
            pltpu.VMEM((CHUNK, OUT), f32),
            pltpu.VMEM((CHUNK, OUT), f32),
            pltpu.VMEM((CHUNK, OUT), f32),
            pltpu.VMEM((CHUNK, OUT), f32),
            pltpu.SemaphoreType.DMA,
            pltpu.SemaphoreType.DMA,
            pltpu.SemaphoreType.DMA,
            pltpu.SemaphoreType.DMA,
            pltpu.SemaphoreType.DMA,
            pltpu.SemaphoreType.DMA,
            pltpu.SemaphoreType.DMA,
            pltpu.SemaphoreType.DMA,
            pltpu.SemaphoreType.DMA,
        ],
    )(q, hef, ni, hi)


def _alphaval_body(qg, kg, mq, mh, g, gt, val, se):
    i = pl.program_id(0)
    kgv = kg[...]
    a = jnp.dot(qg[...] * kgv, g[...],
                preferred_element_type=f32) * (1.0 / math.sqrt(DK))
    bound = mq[...] * mh[...] * (1.0 / math.sqrt(DK)) + 1.0
    w = jnp.exp(a - bound)
    w128 = jnp.dot(w, gt[...], preferred_element_type=f32)
    val[...] = kgv * w128
    prev = jnp.where(i == 0, jnp.zeros((1, OUT), f32), se[...])
    se[...] = prev + jnp.sum(w128, axis=0, keepdims=True)


def _alphaval(qg, kg, mq, mh, g, gt):
    blk = 2000
    io = pl.BlockSpec((blk, OUT), lambda i: (i, 0))
    m = pl.BlockSpec((1, HEADS), lambda i: (0, 0))
    return pl.pallas_call(
        _alphaval_body, grid=(E // blk,),
        in_specs=[io, io, m, m,
                  pl.BlockSpec((OUT, HEADS), lambda i: (0, 0)),
                  pl.BlockSpec((HEADS, OUT), lambda i: (0, 0))],
        out_specs=[pl.BlockSpec((blk, OUT), lambda i: (i, 0)),
                   pl.BlockSpec((1, OUT), lambda i: (0, 0))],
        out_shape=[jax.ShapeDtypeStruct((E, OUT), f32),
                   jax.ShapeDtypeStruct((1, OUT), f32)],
    )(qg, kg, mq, mh, g, gt)


def _sc_nodeout_body(val, ni, zeros, out,
                     nv0, nv1, rv0, rv1, acc,
                     sn0, sn1, sv0, sv1):
    cid = lax.axis_index("c")
    sid = lax.axis_index("s")
    wid = sid * NC + cid
    ebase = pl.multiple_of(wid * EPW, CHUNK)
    nch = jnp.where(wid < 2, NFULL + 1, NFULL)
    _per_sub_rows(sid, lambda off, sz: pltpu.sync_copy(
        zeros.at[pl.ds(off, sz)], acc.at[pl.ds(off, sz)]))
    plsc.subcore_barrier()

    def issue(c, nv, rv, sn, sv):
        off = pl.multiple_of(
            jnp.where(c < NFULL, ebase + c * CHUNK,
                      TAIL + wid * CHUNK), CHUNK)
        pltpu.async_copy(ni.at[pl.ds(off, CHUNK)], nv, sn)
        pltpu.async_copy(val.at[pl.ds(off, CHUNK)], rv, sv)

    def finish(c, nv, rv, sn, sv):
        pltpu.make_async_copy(ni.at[pl.ds(0, CHUNK)], nv, sn).wait()
        pltpu.make_async_copy(val.at[pl.ds(0, CHUNK)], rv, sv).wait()
        pltpu.sync_copy(rv, acc.at[nv], add=True)

    issue(jnp.int32(0), nv0, rv0, sn0, sv0)

    @pl.loop(0, (NFULL + 2) // 2)
    def _(j):
        a = 2 * j
        b2 = 2 * j + 1

        @pl.when(a < nch)
        def _():
            @pl.when(b2 < nch)
            def _():
                issue(b2, nv1, rv1, sn1, sv1)
            finish(a, nv0, rv0, sn0, sv0)

        @pl.when(b2 < nch)
        def _():
            @pl.when(b2 + 1 < nch)
            def _():
                issue(b2 + 1, nv0, rv0, sn0, sv0)
            finish(b2, nv1, rv1, sn1, sv1)
    plsc.subcore_barrier()
    _per_sub_rows(sid, lambda off, sz: pltpu.sync_copy(
        acc.at[pl.ds(off, sz)], out.at[cid, pl.ds(off, sz)]))


def _sc_nodeout(val, ni, zeros):
    return pl.kernel(
        _sc_nodeout_body,
        out_type=jax.ShapeDtypeStruct((NC, N, OUT), f32),
        mesh=_mesh(),
        scratch_types=[
            pltpu.VMEM((CHUNK,), jnp.int32),
            pltpu.VMEM((CHUNK,), jnp.int32),
            pltpu.VMEM((CHUNK, OUT), f32),
            pltpu.VMEM((CHUNK, OUT), f32),
            pltpu.VMEM_SHARED((N, OUT), f32),
            pltpu.SemaphoreType.DMA,
            pltpu.SemaphoreType.DMA,
            pltpu.SemaphoreType.DMA,
            pltpu.SemaphoreType.DMA,
        ],
    )(val, ni, zeros)


def _post_body(parts, se, skip, aw, ab, g, b, o):
    rec = 1.0 / se[...]
    no = (parts[0] + parts[1]) * rec
    merged = jnp.dot(no, aw[...], preferred_element_type=f32) + ab[...]
    y = merged + skip[...]
    mu = jnp.mean(y, axis=-1, keepdims=True)
    var = jnp.mean((y - mu) ** 2, axis=-1, keepdims=True)
    o[...] = (y - mu) * lax.rsqrt(var + 1e-5) * g[...] + b[...]


def _post(parts, se, skip, aw, ab, g, b):
    blk = 2000
    return pl.pallas_call(
        _post_body, grid=(N // blk,),
        in_specs=[pl.BlockSpec((NC, blk, OUT), lambda i: (0, i, 0)),
                  pl.BlockSpec((1, OUT), lambda i: (0, 0)),
                  pl.BlockSpec((blk, OUT), lambda i: (i, 0)),
                  pl.BlockSpec((OUT, OUT), lambda i: (0, 0)),
                  pl.BlockSpec((1, OUT), lambda i: (0, 0)),
                  pl.BlockSpec((1, OUT), lambda i: (0, 0)),
                  pl.BlockSpec((1, OUT), lambda i: (0, 0))],
        out_specs=pl.BlockSpec((blk, OUT), lambda i: (i, 0)),
        out_shape=jax.ShapeDtypeStruct((N, OUT), f32),
    )(parts, se, skip, aw, ab, g, b)


def kernel(x_n0, x_n1, he_index_n0, he_index_n1, max_he_id,
           k_W_n0, k_b_n0, q_W_n0, q_b_n0, a_W_n0, a_b_n0,
           skip_W_n0, skip_b_n0, ln_g_n0, ln_b_n0,
           k_W_n1, k_b_n1, q_W_n1, q_b_n1, a_W_n1, a_b_n1,
           skip_W_n1, skip_b_n1, ln_g_n1, ln_b_n1):
    del max_he_id

    ni0, hi0 = he_index_n0[0], he_index_n0[1]
    ni1, hi1 = he_index_n1[0], he_index_n1[1]

    wcat0 = jnp.concatenate([k_W_n0, q_W_n0, skip_W_n0], axis=1)
    bcat0 = jnp.concatenate([k_b_n0, q_b_n0, skip_b_n0])[None, :]
    wcat1 = jnp.concatenate([k_W_n1, q_W_n1, skip_W_n1], axis=1)
    bcat1 = jnp.concatenate([k_b_n1, q_b_n1, skip_b_n1])[None, :]

    eye = jnp.eye(HEADS, dtype=f32)
    g = jnp.repeat(eye, DK, axis=0)
    gt = jnp.repeat(eye, DK, axis=1)
    zeros = jnp.zeros((N, OUT), f32)

    hk0, q0, s0, hk1, q1, s1 = _dense_pre(x_n0, x_n1, wcat0, bcat0, wcat1, bcat1)

    hef_parts = _sc_hef(hk0, hk1, ni0, hi0, ni1, hi1, zeros)
    hef, mq0, mq1, mh = _combine_norms(hef_parts, q0, q1, g)

    qg0, kg0 = _sc_qk(q0, hef, ni0, hi0)
    qg1, kg1 = _sc_qk(q1, hef, ni1, hi1)
    val0, se0 = _alphaval(qg0, kg0, mq0, mh, g, gt)
    val1, se1 = _alphaval(qg1, kg1, mq1, mh, g, gt)

    parts0 = _sc_nodeout(val0, ni0, zeros)
    parts1 = _sc_nodeout(val1, ni1, zeros)

    out0 = _post(parts0, se0, s0, a_W_n0, a_b_n0[None, :],
                 ln_g_n0[None, :], ln_b_n0[None, :])
    out1 = _post(parts1, se1, s1, a_W_n1, a_b_n1[None, :],
                 ln_g_n1[None, :], ln_b_n1[None, :])
    return (out0, out1)


# speedup vs baseline: 112.1017x; 1.0093x over previous
import functools
import math

import jax
import jax.numpy as jnp
from jax import lax
from jax.experimental import pallas as pl
from jax.experimental.pallas import tpu as pltpu
from jax.experimental.pallas import tpu_sc as plsc

N = 10000
D = 128
OUT = 128
HEADS = 8
DK = OUT // HEADS
E = 160000
NC = 2
NS = 16
NW = NC * NS
CHUNK = 128
NCHUNK = E // CHUNK
SLAB = 640


def _per_sub_rows(sid, copy_fn):
    @pl.when(sid < NS - 1)
    def _():
        copy_fn(pl.multiple_of(sid * SLAB, SLAB), SLAB)

    @pl.when(sid == NS - 1)
    def _():
        copy_fn((NS - 1) * SLAB, N - (NS - 1) * SLAB)
@functools.cache
def _mesh():
    return plsc.VectorSubcoreMesh(core_axis_name="c", subcore_axis_name="s",
                                  num_cores=NC, num_subcores=NS)

f32 = jnp.float32


def _dense_hk_body(x0, x1, w0, b0, w1, b1, hk0, hk1):
    hk0[...] = jnp.dot(x0[...], w0[...], preferred_element_type=f32) + b0[...]
    hk1[...] = jnp.dot(x1[...], w1[...], preferred_element_type=f32) + b1[...]


def _dense_hk(x0, x1, w0, b0, w1, b1):
    blk = 2000
    io = pl.BlockSpec((blk, D), lambda i: (i, 0))
    w = pl.BlockSpec((D, OUT), lambda i: (0, 0))
    b = pl.BlockSpec((1, OUT), lambda i: (0, 0))
    o = pl.BlockSpec((blk, OUT), lambda i: (i, 0))
    sh = jax.ShapeDtypeStruct((N, OUT), f32)
    return pl.pallas_call(
        _dense_hk_body, grid=(N // blk,),
        in_specs=[io, io, w, b, w, b],
        out_specs=[o, o],
        out_shape=[sh, sh],
    )(x0, x1, w0, b0, w1, b1)


def _dense_qs_body(x0, x1, w0, b0, w1, b1, q0, s0, q1, s1):
    r0 = jnp.dot(x0[...], w0[...], preferred_element_type=f32) + b0[...]
    q0[...] = r0[:, 0:OUT]
    s0[...] = r0[:, OUT:2 * OUT]
    r1 = jnp.dot(x1[...], w1[...], preferred_element_type=f32) + b1[...]
    q1[...] = r1[:, 0:OUT]
    s1[...] = r1[:, OUT:2 * OUT]


def _dense_qs(x0, x1, w0, b0, w1, b1):
    blk = 2000
    io = pl.BlockSpec((blk, D), lambda i: (i, 0))
    w = pl.BlockSpec((D, 2 * OUT), lambda i: (0, 0))
    b = pl.BlockSpec((1, 2 * OUT), lambda i: (0, 0))
    o = pl.BlockSpec((blk, OUT), lambda i: (i, 0))
    shf = jax.ShapeDtypeStruct((N, OUT), f32)
    return pl.pallas_call(
        _dense_qs_body, grid=(N // blk,),
        in_specs=[io, io, w, b, w, b],
        out_specs=[o] * 4,
        out_shape=[shf, shf, shf, shf],
    )(x0, x1, w0, b0, w1, b1)


EPW = 4992
NFULL = EPW // CHUNK
TAIL = NFULL * CHUNK * NW


def _sc_hef_body(hk0, hk1, ni0, hi0, ni1, hi1, zeros, out,
                 nvf, nve, hv0, hv1, rows0, rows1, acc,
                 sg0, sg1, sh0, sh1, ss):
    cid = lax.axis_index("c")
    sid = lax.axis_index("s")
    wid = sid * NC + cid
    ebase = pl.multiple_of(wid * EPW, CHUNK)
    nch = jnp.where(wid < 2, NFULL + 1, NFULL)
    _per_sub_rows(sid, lambda off, sz: pltpu.sync_copy(
        zeros.at[pl.ds(off, sz)], acc.at[pl.ds(off, sz)]))
    plsc.subcore_barrier()
    for hk, ni, hi in ((hk0, ni0, hi0), (hk1, ni1, hi1)):
        pltpu.async_copy(ni.at[pl.ds(ebase, EPW)], nvf, ss).wait()

        @pl.when(wid < 2)
        def _():
            pltpu.sync_copy(ni.at[pl.ds(pl.multiple_of(TAIL + wid * CHUNK,
                                                       CHUNK), CHUNK)], nve)

        def issue(c, hv, rows, sg, sh):
            @pl.when(c < NFULL)
            def _():
                b = pl.multiple_of(c * CHUNK, CHUNK)
                pltpu.async_copy(hi.at[pl.ds(pl.multiple_of(ebase + b, CHUNK),
                                             CHUNK)], hv, sh)
                pltpu.async_copy(hk.at[nvf.at[pl.ds(b, CHUNK)]], rows, sg)

            @pl.when(c == NFULL)
            def _():
                off = pl.multiple_of(TAIL + wid * CHUNK, CHUNK)
                pltpu.async_copy(hi.at[pl.ds(off, CHUNK)], hv, sh)
                pltpu.async_copy(hk.at[nve], rows, sg)

        def finish(c, hv, rows, sg, sh):
            pltpu.make_async_copy(hi.at[pl.ds(0, CHUNK)], hv, sh).wait()
            pltpu.make_async_copy(hk.at[nve], rows, sg).wait()
            pltpu.sync_copy(rows, acc.at[hv], add=True)

        issue(0, hv0, rows0, sg0, sh0)

        @pl.loop(0, (NFULL + 2) // 2)
        def _(j):
            a = 2 * j
            b2 = 2 * j + 1

            @pl.when(a < nch)
            def _():
                @pl.when(b2 < nch)
                def _():
                    issue(b2, hv1, rows1, sg1, sh1)
                finish(a, hv0, rows0, sg0, sh0)

            @pl.when(b2 < nch)
            def _():
                @pl.when(b2 + 1 < nch)
                def _():
                    issue(b2 + 1, hv0, rows0, sg0, sh0)
                finish(b2, hv1, rows1, sg1, sh1)
    plsc.subcore_barrier()
    _per_sub_rows(sid, lambda off, sz: pltpu.sync_copy(
        acc.at[pl.ds(off, sz)], out.at[cid, pl.ds(off, sz)]))


def _sc_hef(hk0, hk1, ni0, hi0, ni1, hi1, zeros):
    return pl.kernel(
        _sc_hef_body,
        out_type=jax.ShapeDtypeStruct((NC, N, OUT), f32),
        mesh=_mesh(),
        scratch_types=[
            pltpu.VMEM((EPW,), jnp.int32),
            pltpu.VMEM((CHUNK,), jnp.int32),
            pltpu.VMEM((CHUNK,), jnp.int32),
            pltpu.VMEM((CHUNK,), jnp.int32),
            pltpu.VMEM((CHUNK, OUT), f32),
            pltpu.VMEM((CHUNK, OUT), f32),
            pltpu.VMEM_SHARED((N, OUT), f32),
            pltpu.SemaphoreType.DMA,
            pltpu.SemaphoreType.DMA,
            pltpu.SemaphoreType.DMA,
            pltpu.SemaphoreType.DMA,
            pltpu.SemaphoreType.DMA,
        ],
    )(hk0, hk1, ni0, hi0, ni1, hi1, zeros)


def _combine_norms_body(p, q0, q1, g, hef, mq0, mq1, mh):
    i = pl.program_id(0)
    h = p[0] + p[1]
    hef[...] = h

    def _nmax(x):
        n = jnp.sqrt(jnp.dot(x * x, g[...], preferred_element_type=f32))
        return jnp.max(n, axis=0, keepdims=True)

    neg = jnp.full((1, HEADS), -jnp.inf, f32)
    for ref, x in ((mq0, q0[...]), (mq1, q1[...]), (mh, h)):
        prev = jnp.where(i == 0, neg, ref[...])
        ref[...] = jnp.maximum(prev, _nmax(x))


def _combine_norms(parts, q0, q1, g):
    blk = 2000
    io = pl.BlockSpec((blk, OUT), lambda i: (i, 0))
    m = pl.BlockSpec((1, HEADS), lambda i: (0, 0))
    msh = jax.ShapeDtypeStruct((1, HEADS), f32)
    return pl.pallas_call(
        _combine_norms_body, grid=(N // blk,),
        in_specs=[pl.BlockSpec((NC, blk, OUT), lambda i: (0, i, 0)), io, io,
                  pl.BlockSpec((OUT, HEADS), lambda i: (0, 0))],
        out_specs=[pl.BlockSpec((blk, OUT), lambda i: (i, 0)), m, m, m],
        out_shape=[jax.ShapeDtypeStruct((N, OUT), f32), msh, msh, msh],
    )(parts, q0, q1, g)


def _sc_qk_body(q, hef, ni, hi,
                qg, kg,
                nvf, hvf, nve, hve, qr0, qr1, kr0, kr1,
                sq0, sq1, sk0, sk1, wq0, wq1, wk0, wk1, ss):
    cid = lax.axis_index("c")
    sid = lax.axis_index("s")
    wid = sid * NC + cid
    ebase = pl.multiple_of(wid * EPW, CHUNK)
    nch = jnp.where(wid < 2, NFULL + 1, NFULL)
    cpa = pltpu.async_copy(ni.at[pl.ds(ebase, EPW)], nvf, ss)
    cpb = pltpu.async_copy(hi.at[pl.ds(ebase, EPW)], hvf, ss)
    cpa.wait()
    cpb.wait()

    @pl.when(wid < 2)
    def _():
        toff = pl.multiple_of(TAIL + wid * CHUNK, CHUNK)
        pltpu.sync_copy(ni.at[pl.ds(toff, CHUNK)], nve)
        pltpu.sync_copy(hi.at[pl.ds(toff, CHUNK)], hve)

    def pre(c, qr, kr, sq, sk, wq, wk):
        @pl.when(c < nch)
        def _():
            @pl.when(c >= 2)
            def _():
                pltpu.make_async_copy(qr, qg.at[pl.ds(0, CHUNK)], wq).wait()
                pltpu.make_async_copy(kr, kg.at[pl.ds(0, CHUNK)], wk).wait()

            @pl.when(c < NFULL)
            def _():
                b = pl.multiple_of(c * CHUNK, CHUNK)
                pltpu.async_copy(q.at[nvf.at[pl.ds(b, CHUNK)]], qr, sq)
                pltpu.async_copy(hef.at[hvf.at[pl.ds(b, CHUNK)]], kr, sk)

            @pl.when(c == NFULL)
            def _():
                pltpu.async_copy(q.at[nve], qr, sq)
                pltpu.async_copy(hef.at[hve], kr, sk)

    def proc(c, qr, kr, sq, sk, wq, wk):
        @pl.when(c < nch)
        def _():
            pltpu.make_async_copy(q.at[nve], qr, sq).wait()
            pltpu.make_async_copy(hef.at[hve], kr, sk).wait()
            woff = pl.multiple_of(
                jnp.where(c < NFULL, ebase + c * CHUNK,
                          TAIL + wid * CHUNK), CHUNK)
            pltpu.async_copy(qr, qg.at[pl.ds(woff, CHUNK)], wq)
            pltpu.async_copy(kr, kg.at[pl.ds(woff, CHUNK)], wk)

    pre(jnp.int32(0), qr0, kr0, sq0, sk0, wq0, wk0)
    pre(jnp.int32(1), qr1, kr1, sq1, sk1, wq1, wk1)

    @pl.loop(0, (NFULL + 2) // 2)
    def _(j):
        a = 2 * j
        b2 = 2 * j + 1
        proc(a, qr0, kr0, sq0, sk0, wq0, wk0)
        pre(a + 2, qr0, kr0, sq0, sk0, wq0, wk0)
        proc(b2, qr1, kr1, sq1, sk1, wq1, wk1)
        pre(b2 + 2, qr1, kr1, sq1, sk1, wq1, wk1)

    pltpu.make_async_copy(qr0, qg.at[pl.ds(0, CHUNK)], wq0).wait()
    pltpu.make_async_copy(kr0, kg.at[pl.ds(0, CHUNK)], wk0).wait()
    pltpu.make_async_copy(qr1, qg.at[pl.ds(0, CHUNK)], wq1).wait()
    pltpu.make_async_copy(kr1, kg.at[pl.ds(0, CHUNK)], wk1).wait()


def _sc_qk(q, hef, ni, hi):
    sh = jax.ShapeDtypeStruct((E, OUT), f32)
    return pl.kernel(
        _sc_qk_body,
        out_type=(sh, sh),
        mesh=_mesh(),
        scratch_types=[
            pltpu.VMEM((EPW,), jnp.int32),
            pltpu.VMEM((EPW,), jnp.int32),
            pltpu.VMEM((CHUNK,), jnp.int32),
            pltpu.VMEM((CHUNK,), jnp.int32),
            pltpu.VMEM((CHUNK, OUT), f32),
            pltpu.VMEM((CHUNK, OUT), f32),
            pltpu.VMEM((CHUNK, OUT), f32),
            pltpu.VMEM((CHUNK, OUT), f32),
            pltpu.SemaphoreType.DMA,
            pltpu.SemaphoreType.DMA,
            pltpu.SemaphoreType.DMA,
            pltpu.SemaphoreType.DMA,
            pltpu.SemaphoreType.DMA,
            pltpu.SemaphoreType.DMA,
            pltpu.SemaphoreType.DMA,
            pltpu.SemaphoreType.DMA,
            pltpu.SemaphoreType.DMA,
        ],
    )(q, hef, ni, hi)


def _alphaval_body(qg, kg, mq, mh, g, gt, val, se):
    i = pl.program_id(0)
    kgv = kg[...]
    a = jnp.dot(qg[...] * kgv, g[...],
                preferred_element_type=f32) * (1.0 / math.sqrt(DK))
    bound = mq[...] * mh[...] * (1.0 / math.sqrt(DK)) + 1.0
    w = jnp.exp(a - bound)
    w128 = jnp.dot(w, gt[...], preferred_element_type=f32)
    val[...] = kgv * w128
    prev = jnp.where(i == 0, jnp.zeros((1, OUT), f32), se[...])
    se[...] = prev + jnp.sum(w128, axis=0, keepdims=True)


def _alphaval(qg, kg, mq, mh, g, gt):
    blk = 2000
    io = pl.BlockSpec((blk, OUT), lambda i: (i, 0))
    m = pl.BlockSpec((1, HEADS), lambda i: (0, 0))
    return pl.pallas_call(
        _alphaval_body, grid=(E // blk,),
        in_specs=[io, io, m, m,
                  pl.BlockSpec((OUT, HEADS), lambda i: (0, 0)),
                  pl.BlockSpec((HEADS, OUT), lambda i: (0, 0))],
        out_specs=[pl.BlockSpec((blk, OUT), lambda i: (i, 0)),
                   pl.BlockSpec((1, OUT), lambda i: (0, 0))],
        out_shape=[jax.ShapeDtypeStruct((E, OUT), f32),
                   jax.ShapeDtypeStruct((1, OUT), f32)],
    )(qg, kg, mq, mh, g, gt)


def _sc_nodeout_body(val, ni, zeros, out,
                     nv0, nv1, rv0, rv1, acc,
                     sn0, sn1, sv0, sv1):
    cid = lax.axis_index("c")
    sid = lax.axis_index("s")
    wid = sid * NC + cid
    ebase = pl.multiple_of(wid * EPW, CHUNK)
    nch = jnp.where(wid < 2, NFULL + 1, NFULL)
    _per_sub_rows(sid, lambda off, sz: pltpu.sync_copy(
        zeros.at[pl.ds(off, sz)], acc.at[pl.ds(off, sz)]))
    plsc.subcore_barrier()

    def issue(c, nv, rv, sn, sv):
        off = pl.multiple_of(
            jnp.where(c < NFULL, ebase + c * CHUNK,
                      TAIL + wid * CHUNK), CHUNK)
        pltpu.async_copy(ni.at[pl.ds(off, CHUNK)], nv, sn)
        pltpu.async_copy(val.at[pl.ds(off, CHUNK)], rv, sv)

    def finish(c, nv, rv, sn, sv):
        pltpu.make_async_copy(ni.at[pl.ds(0, CHUNK)], nv, sn).wait()
        pltpu.make_async_copy(val.at[pl.ds(0, CHUNK)], rv, sv).wait()
        pltpu.sync_copy(rv, acc.at[nv], add=True)

    issue(jnp.int32(0), nv0, rv0, sn0, sv0)

    @pl.loop(0, (NFULL + 2) // 2)
    def _(j):
        a = 2 * j
        b2 = 2 * j + 1

        @pl.when(a < nch)
        def _():
            @pl.when(b2 < nch)
            def _():
                issue(b2, nv1, rv1, sn1, sv1)
            finish(a, nv0, rv0, sn0, sv0)

        @pl.when(b2 < nch)
        def _():
            @pl.when(b2 + 1 < nch)
            def _():
                issue(b2 + 1, nv0, rv0, sn0, sv0)
            finish(b2, nv1, rv1, sn1, sv1)
    plsc.subcore_barrier()
    _per_sub_rows(sid, lambda off, sz: pltpu.sync_copy(
        acc.at[pl.ds(off, sz)], out.at[cid, pl.ds(off, sz)]))


def _sc_nodeout(val, ni, zeros):
    return pl.kernel(
        _sc_nodeout_body,
        out_type=jax.ShapeDtypeStruct((NC, N, OUT), f32),
        mesh=_mesh(),
        scratch_types=[
            pltpu.VMEM((CHUNK,), jnp.int32),
            pltpu.VMEM((CHUNK,), jnp.int32),
            pltpu.VMEM((CHUNK, OUT), f32),
            pltpu.VMEM((CHUNK, OUT), f32),
            pltpu.VMEM_SHARED((N, OUT), f32),
            pltpu.SemaphoreType.DMA,
            pltpu.SemaphoreType.DMA,
            pltpu.SemaphoreType.DMA,
            pltpu.SemaphoreType.DMA,
        ],
    )(val, ni, zeros)


def _post_body(parts, se, skip, aw, ab, g, b, o):
    rec = 1.0 / se[...]
    no = (parts[0] + parts[1]) * rec
    merged = jnp.dot(no, aw[...], preferred_element_type=f32) + ab[...]
    y = merged + skip[...]
    mu = jnp.mean(y, axis=-1, keepdims=True)
    var = jnp.mean((y - mu) ** 2, axis=-1, keepdims=True)
    o[...] = (y - mu) * lax.rsqrt(var + 1e-5) * g[...] + b[...]


def _post(parts, se, skip, aw, ab, g, b):
    blk = 2000
    return pl.pallas_call(
        _post_body, grid=(N // blk,),
        in_specs=[pl.BlockSpec((NC, blk, OUT), lambda i: (0, i, 0)),
                  pl.BlockSpec((1, OUT), lambda i: (0, 0)),
                  pl.BlockSpec((blk, OUT), lambda i: (i, 0)),
                  pl.BlockSpec((OUT, OUT), lambda i: (0, 0)),
                  pl.BlockSpec((1, OUT), lambda i: (0, 0)),
                  pl.BlockSpec((1, OUT), lambda i: (0, 0)),
                  pl.BlockSpec((1, OUT), lambda i: (0, 0))],
        out_specs=pl.BlockSpec((blk, OUT), lambda i: (i, 0)),
        out_shape=jax.ShapeDtypeStruct((N, OUT), f32),
    )(parts, se, skip, aw, ab, g, b)


def kernel(x_n0, x_n1, he_index_n0, he_index_n1, max_he_id,
           k_W_n0, k_b_n0, q_W_n0, q_b_n0, a_W_n0, a_b_n0,
           skip_W_n0, skip_b_n0, ln_g_n0, ln_b_n0,
           k_W_n1, k_b_n1, q_W_n1, q_b_n1, a_W_n1, a_b_n1,
           skip_W_n1, skip_b_n1, ln_g_n1, ln_b_n1):
    del max_he_id

    ni0, hi0 = he_index_n0[0], he_index_n0[1]
    ni1, hi1 = he_index_n1[0], he_index_n1[1]

    wqs0 = jnp.concatenate([q_W_n0, skip_W_n0], axis=1)
    bqs0 = jnp.concatenate([q_b_n0, skip_b_n0])[None, :]
    wqs1 = jnp.concatenate([q_W_n1, skip_W_n1], axis=1)
    bqs1 = jnp.concatenate([q_b_n1, skip_b_n1])[None, :]

    eye = jnp.eye(HEADS, dtype=f32)
    g = jnp.repeat(eye, DK, axis=0)
    gt = jnp.repeat(eye, DK, axis=1)
    zeros = jnp.zeros((N, OUT), f32)

    hk0, hk1 = _dense_hk(x_n0, x_n1, k_W_n0, k_b_n0[None, :],
                         k_W_n1, k_b_n1[None, :])
    hef_parts = _sc_hef(hk0, hk1, ni0, hi0, ni1, hi1, zeros)
    q0, s0, q1, s1 = _dense_qs(x_n0, x_n1, wqs0, bqs0, wqs1, bqs1)
    hef, mq0, mq1, mh = _combine_norms(hef_parts, q0, q1, g)

    qg0, kg0 = _sc_qk(q0, hef, ni0, hi0)
    qg1, kg1 = _sc_qk(q1, hef, ni1, hi1)
    val0, se0 = _alphaval(qg0, kg0, mq0, mh, g, gt)
    val1, se1 = _alphaval(qg1, kg1, mq1, mh, g, gt)

    parts0 = _sc_nodeout(val0, ni0, zeros)
    parts1 = _sc_nodeout(val1, ni1, zeros)

    out0 = _post(parts0, se0, s0, a_W_n0, a_b_n0[None, :],
                 ln_g_n0[None, :], ln_b_n0[None, :])
    out1 = _post(parts1, se1, s1, a_W_n1, a_b_n1[None, :],
                 ln_g_n1[None, :], ln_b_n1[None, :])
    return (out0, out1)


# speedup vs baseline: 112.4733x; 1.0033x over previous
import functools
import math

import jax
import jax.numpy as jnp
from jax import lax
from jax.experimental import pallas as pl
from jax.experimental.pallas import tpu as pltpu
from jax.experimental.pallas import tpu_sc as plsc

N = 10000
D = 128
OUT = 128
HEADS = 8
DK = OUT // HEADS
E = 160000
NC = 2
NS = 16
NW = NC * NS
CHUNK = 128
NCHUNK = E // CHUNK
SLAB = 640


def _per_sub_rows(sid, copy_fn):
    @pl.when(sid < NS - 1)
    def _():
        copy_fn(pl.multiple_of(sid * SLAB, SLAB), SLAB)

    @pl.when(sid == NS - 1)
    def _():
        copy_fn((NS - 1) * SLAB, N - (NS - 1) * SLAB)
@functools.cache
def _mesh():
    return plsc.VectorSubcoreMesh(core_axis_name="c", subcore_axis_name="s",
                                  num_cores=NC, num_subcores=NS)

f32 = jnp.float32


def _dense_hk_body(x0, x1, w0, b0, w1, b1, hk0, hk1):
    hk0[...] = jnp.dot(x0[...], w0[...], preferred_element_type=f32) + b0[...]
    hk1[...] = jnp.dot(x1[...], w1[...], preferred_element_type=f32) + b1[...]


def _dense_hk(x0, x1, w0, b0, w1, b1):
    blk = 2000
    io = pl.BlockSpec((blk, D), lambda i: (i, 0))
    w = pl.BlockSpec((D, OUT), lambda i: (0, 0))
    b = pl.BlockSpec((1, OUT), lambda i: (0, 0))
    o = pl.BlockSpec((blk, OUT), lambda i: (i, 0))
    sh = jax.ShapeDtypeStruct((N, OUT), f32)
    return pl.pallas_call(
        _dense_hk_body, grid=(N // blk,),
        in_specs=[io, io, w, b, w, b],
        out_specs=[o, o],
        out_shape=[sh, sh],
    )(x0, x1, w0, b0, w1, b1)


def _dense_qs_body(x0, x1, w0, b0, w1, b1, q0, s0, q1, s1):
    r0 = jnp.dot(x0[...], w0[...], preferred_element_type=f32) + b0[...]
    q0[...] = r0[:, 0:OUT]
    s0[...] = r0[:, OUT:2 * OUT]
    r1 = jnp.dot(x1[...], w1[...], preferred_element_type=f32) + b1[...]
    q1[...] = r1[:, 0:OUT]
    s1[...] = r1[:, OUT:2 * OUT]


def _dense_qs(x0, x1, w0, b0, w1, b1):
    blk = 2000
    io = pl.BlockSpec((blk, D), lambda i: (i, 0))
    w = pl.BlockSpec((D, 2 * OUT), lambda i: (0, 0))
    b = pl.BlockSpec((1, 2 * OUT), lambda i: (0, 0))
    o = pl.BlockSpec((blk, OUT), lambda i: (i, 0))
    shf = jax.ShapeDtypeStruct((N, OUT), f32)
    return pl.pallas_call(
        _dense_qs_body, grid=(N // blk,),
        in_specs=[io, io, w, b, w, b],
        out_specs=[o] * 4,
        out_shape=[shf, shf, shf, shf],
    )(x0, x1, w0, b0, w1, b1)


EPW = 4992
NFULL = EPW // CHUNK
TAIL = NFULL * CHUNK * NW


def _sc_hef_body(hk0, hk1, ni0, hi0, ni1, hi1, zeros, out,
                 nvf, nve, hv0, hv1, rows0, rows1, acc,
                 sg0, sg1, sh0, sh1, ss):
    cid = lax.axis_index("c")
    sid = lax.axis_index("s")
    wid = sid * NC + cid
    ebase = pl.multiple_of(wid * EPW, CHUNK)
    nch = jnp.where(wid < 2, NFULL + 1, NFULL)
    hv = (hv0, hv1)
    rows = (rows0, rows1)
    sg = (sg0, sg1)
    sh = (sh0, sh1)
    _per_sub_rows(sid, lambda off, sz: pltpu.sync_copy(
        zeros.at[pl.ds(off, sz)], acc.at[pl.ds(off, sz)]))
    plsc.subcore_barrier()
    for hk, ni, hi in ((hk0, ni0, hi0), (hk1, ni1, hi1)):
        pltpu.async_copy(ni.at[pl.ds(ebase, EPW)], nvf, ss).wait()

        @pl.when(wid < 2)
        def _():
            pltpu.sync_copy(ni.at[pl.ds(pl.multiple_of(TAIL + wid * CHUNK,
                                                       CHUNK), CHUNK)], nve)

        def issue(c, p):
            @pl.when(c < NFULL)
            def _():
                b = pl.multiple_of(c * CHUNK, CHUNK)
                pltpu.async_copy(hi.at[pl.ds(pl.multiple_of(ebase + b, CHUNK),
                                             CHUNK)], hv[p], sh[p])
                pltpu.async_copy(hk.at[nvf.at[pl.ds(b, CHUNK)]], rows[p], sg[p])

            @pl.when(c == NFULL)
            def _():
                off = pl.multiple_of(TAIL + wid * CHUNK, CHUNK)
                pltpu.async_copy(hi.at[pl.ds(off, CHUNK)], hv[p], sh[p])
                pltpu.async_copy(hk.at[nve], rows[p], sg[p])

        def finish(c, p):
            @pl.when(c < nch)
            def _():
                pltpu.make_async_copy(hi.at[pl.ds(0, CHUNK)], hv[p], sh[p]).wait()
                pltpu.make_async_copy(hk.at[nve], rows[p], sg[p]).wait()
                pltpu.sync_copy(rows[p], acc.at[hv[p]], add=True)

        for p in range(2):
            issue(jnp.int32(p), p)

        @pl.loop(0, (NFULL + 2) // 2)
        def _(j):
            for p in range(2):
                c = 2 * j + p
                finish(c, p)

                @pl.when(c + 2 < nch)
                def _():
                    issue(c + 2, p)
    plsc.subcore_barrier()
    _per_sub_rows(sid, lambda off, sz: pltpu.sync_copy(
        acc.at[pl.ds(off, sz)], out.at[cid, pl.ds(off, sz)]))


def _sc_hef(hk0, hk1, ni0, hi0, ni1, hi1, zeros):
    return pl.kernel(
        _sc_hef_body,
        out_type=jax.ShapeDtypeStruct((NC, N, OUT), f32),
        mesh=_mesh(),
        scratch_types=[
            pltpu.VMEM((EPW,), jnp.int32),
            pltpu.VMEM((CHUNK,), jnp.int32),
            pltpu.VMEM((CHUNK,), jnp.int32),
            pltpu.VMEM((CHUNK,), jnp.int32),
            pltpu.VMEM((CHUNK, OUT), f32),
            pltpu.VMEM((CHUNK, OUT), f32),
            pltpu.VMEM_SHARED((N, OUT), f32),
        ] + [pltpu.SemaphoreType.DMA] * 5,
    )(hk0, hk1, ni0, hi0, ni1, hi1, zeros)


def _combine_norms_body(p, q0, q1, g, hef, mq0, mq1, mh):
    i = pl.program_id(0)
    h = p[0] + p[1]
    hef[...] = h

    def _nmax(x):
        n = jnp.sqrt(jnp.dot(x * x, g[...], preferred_element_type=f32))
        return jnp.max(n, axis=0, keepdims=True)

    neg = jnp.full((1, HEADS), -jnp.inf, f32)
    for ref, x in ((mq0, q0[...]), (mq1, q1[...]), (mh, h)):
        prev = jnp.where(i == 0, neg, ref[...])
        ref[...] = jnp.maximum(prev, _nmax(x))


def _combine_norms(parts, q0, q1, g):
    blk = 2000
    io = pl.BlockSpec((blk, OUT), lambda i: (i, 0))
    m = pl.BlockSpec((1, HEADS), lambda i: (0, 0))
    msh = jax.ShapeDtypeStruct((1, HEADS), f32)
    return pl.pallas_call(
        _combine_norms_body, grid=(N // blk,),
        in_specs=[pl.BlockSpec((NC, blk, OUT), lambda i: (0, i, 0)), io, io,
                  pl.BlockSpec((OUT, HEADS), lambda i: (0, 0))],
        out_specs=[pl.BlockSpec((blk, OUT), lambda i: (i, 0)), m, m, m],
        out_shape=[jax.ShapeDtypeStruct((N, OUT), f32), msh, msh, msh],
    )(parts, q0, q1, g)


def _sc_qk_body(q, hef, ni, hi,
                qg, kg,
                nvf, hvf, nve, hve,
                qr0, qr1, qr2, kr0, kr1, kr2,
                sq0, sq1, sq2, sk0, sk1, sk2,
                wq0, wq1, wq2, wk0, wk1, wk2, ss):
    cid = lax.axis_index("c")
    sid = lax.axis_index("s")
    wid = sid * NC + cid
    ebase = pl.multiple_of(wid * EPW, CHUNK)
    nch = jnp.where(wid < 2, NFULL + 1, NFULL)
    qr = (qr0, qr1, qr2)
    kr = (kr0, kr1, kr2)
    sq = (sq0, sq1, sq2)
    sk = (sk0, sk1, sk2)
    wq = (wq0, wq1, wq2)
    wk = (wk0, wk1, wk2)
    cpa = pltpu.async_copy(ni.at[pl.ds(ebase, EPW)], nvf, ss)
    cpb = pltpu.async_copy(hi.at[pl.ds(ebase, EPW)], hvf, ss)
    cpa.wait()
    cpb.wait()

    @pl.when(wid < 2)
    def _():
        toff = pl.multiple_of(TAIL + wid * CHUNK, CHUNK)
        pltpu.sync_copy(ni.at[pl.ds(toff, CHUNK)], nve)
        pltpu.sync_copy(hi.at[pl.ds(toff, CHUNK)], hve)

    def pre(c, p):
        @pl.when(c < nch)
        def _():
            @pl.when(c >= 3)
            def _():
                pltpu.make_async_copy(qr[p], qg.at[pl.ds(0, CHUNK)], wq[p]).wait()
                pltpu.make_async_copy(kr[p], kg.at[pl.ds(0, CHUNK)], wk[p]).wait()

            @pl.when(c < NFULL)
            def _():
                b = pl.multiple_of(c * CHUNK, CHUNK)
                pltpu.async_copy(q.at[nvf.at[pl.ds(b, CHUNK)]], qr[p], sq[p])
                pltpu.async_copy(hef.at[hvf.at[pl.ds(b, CHUNK)]], kr[p], sk[p])

            @pl.when(c == NFULL)
            def _():
                pltpu.async_copy(q.at[nve], qr[p], sq[p])
                pltpu.async_copy(hef.at[hve], kr[p], sk[p])

    def proc(c, p):
        @pl.when(c < nch)
        def _():
            pltpu.make_async_copy(q.at[nve], qr[p], sq[p]).wait()
            pltpu.make_async_copy(hef.at[hve], kr[p], sk[p]).wait()
            woff = pl.multiple_of(
                jnp.where(c < NFULL, ebase + c * CHUNK,
                          TAIL + wid * CHUNK), CHUNK)
            pltpu.async_copy(qr[p], qg.at[pl.ds(woff, CHUNK)], wq[p])
            pltpu.async_copy(kr[p], kg.at[pl.ds(woff, CHUNK)], wk[p])

    for p in range(3):
        pre(jnp.int32(p), p)

    @pl.loop(0, (NFULL + 3) // 3)
    def _(j):
        for p in range(3):
            c = 3 * j + p
            proc(c, p)
            pre(c + 3, p)

    for p in range(3):
        pltpu.make_async_copy(qr[p], qg.at[pl.ds(0, CHUNK)], wq[p]).wait()
        pltpu.make_async_copy(kr[p], kg.at[pl.ds(0, CHUNK)], wk[p]).wait()


def _sc_qk(q, hef, ni, hi):
    sh = jax.ShapeDtypeStruct((E, OUT), f32)
    return pl.kernel(
        _sc_qk_body,
        out_type=(sh, sh),
        mesh=_mesh(),
        scratch_types=[
            pltpu.VMEM((EPW,), jnp.int32),
            pltpu.VMEM((EPW,), jnp.int32),
            pltpu.VMEM((CHUNK,), jnp.int32),
            pltpu.VMEM((CHUNK,), jnp.int32),
            pltpu.VMEM((CHUNK, OUT), f32),
            pltpu.VMEM((CHUNK, OUT), f32),
            pltpu.VMEM((CHUNK, OUT), f32),
            pltpu.VMEM((CHUNK, OUT), f32),
            pltpu.VMEM((CHUNK, OUT), f32),
            pltpu.VMEM((CHUNK, OUT), f32),
        ] + [pltpu.SemaphoreType.DMA] * 13,
    )(q, hef, ni, hi)


def _alphaval_body(qg, kg, mq, mh, g, gt, val, se):
    i = pl.program_id(0)
    kgv = kg[...]
    a = jnp.dot(qg[...] * kgv, g[...],
                preferred_element_type=f32) * (1.0 / math.sqrt(DK))
    bound = mq[...] * mh[...] * (1.0 / math.sqrt(DK)) + 1.0
    w = jnp.exp(a - bound)
    w128 = jnp.dot(w, gt[...], preferred_element_type=f32)
    val[...] = kgv * w128
    prev = jnp.where(i == 0, jnp.zeros((1, OUT), f32), se[...])
    se[...] = prev + jnp.sum(w128, axis=0, keepdims=True)


def _alphaval(qg, kg, mq, mh, g, gt):
    blk = 2000
    io = pl.BlockSpec((blk, OUT), lambda i: (i, 0))
    m = pl.BlockSpec((1, HEADS), lambda i: (0, 0))
    return pl.pallas_call(
        _alphaval_body, grid=(E // blk,),
        in_specs=[io, io, m, m,
                  pl.BlockSpec((OUT, HEADS), lambda i: (0, 0)),
                  pl.BlockSpec((HEADS, OUT), lambda i: (0, 0))],
        out_specs=[pl.BlockSpec((blk, OUT), lambda i: (i, 0)),
                   pl.BlockSpec((1, OUT), lambda i: (0, 0))],
        out_shape=[jax.ShapeDtypeStruct((E, OUT), f32),
                   jax.ShapeDtypeStruct((1, OUT), f32)],
    )(qg, kg, mq, mh, g, gt)


def _sc_nodeout_body(val, ni, zeros, out,
                     nv0, nv1, nv2, rv0, rv1, rv2, acc,
                     sn0, sn1, sn2, sv0, sv1, sv2):
    cid = lax.axis_index("c")
    sid = lax.axis_index("s")
    wid = sid * NC + cid
    ebase = pl.multiple_of(wid * EPW, CHUNK)
    nch = jnp.where(wid < 2, NFULL + 1, NFULL)
    nv = (nv0, nv1, nv2)
    rv = (rv0, rv1, rv2)
    sn = (sn0, sn1, sn2)
    sv = (sv0, sv1, sv2)
    _per_sub_rows(sid, lambda off, sz: pltpu.sync_copy(
        zeros.at[pl.ds(off, sz)], acc.at[pl.ds(off, sz)]))
    plsc.subcore_barrier()

    def issue(c, p):
        off = pl.multiple_of(
            jnp.where(c < NFULL, ebase + c * CHUNK,
                      TAIL + wid * CHUNK), CHUNK)
        pltpu.async_copy(ni.at[pl.ds(off, CHUNK)], nv[p], sn[p])
        pltpu.async_copy(val.at[pl.ds(off, CHUNK)], rv[p], sv[p])

    def finish(c, p):
        @pl.when(c < nch)
        def _():
            pltpu.make_async_copy(ni.at[pl.ds(0, CHUNK)], nv[p], sn[p]).wait()
            pltpu.make_async_copy(val.at[pl.ds(0, CHUNK)], rv[p], sv[p]).wait()
            pltpu.sync_copy(rv[p], acc.at[nv[p]], add=True)

    for p in range(3):
        issue(jnp.int32(p), p)

    @pl.loop(0, (NFULL + 3) // 3)
    def _(j):
        for p in range(3):
            c = 3 * j + p
            finish(c, p)

            @pl.when(c + 3 < nch)
            def _():
                issue(c + 3, p)
    plsc.subcore_barrier()
    _per_sub_rows(sid, lambda off, sz: pltpu.sync_copy(
        acc.at[pl.ds(off, sz)], out.at[cid, pl.ds(off, sz)]))


def _sc_nodeout(val, ni, zeros):
    return pl.kernel(
        _sc_nodeout_body,
        out_type=jax.ShapeDtypeStruct((NC, N, OUT), f32),
        mesh=_mesh(),
        scratch_types=[
            pltpu.VMEM((CHUNK,), jnp.int32),
            pltpu.VMEM((CHUNK,), jnp.int32),
            pltpu.VMEM((CHUNK,), jnp.int32),
            pltpu.VMEM((CHUNK, OUT), f32),
            pltpu.VMEM((CHUNK, OUT), f32),
            pltpu.VMEM((CHUNK, OUT), f32),
            pltpu.VMEM_SHARED((N, OUT), f32),
        ] + [pltpu.SemaphoreType.DMA] * 6,
    )(val, ni, zeros)


def _post_body(parts, se, skip, aw, ab, g, b, o):
    rec = 1.0 / se[...]
    no = (parts[0] + parts[1]) * rec
    merged = jnp.dot(no, aw[...], preferred_element_type=f32) + ab[...]
    y = merged + skip[...]
    mu = jnp.mean(y, axis=-1, keepdims=True)
    var = jnp.mean((y - mu) ** 2, axis=-1, keepdims=True)
    o[...] = (y - mu) * lax.rsqrt(var + 1e-5) * g[...] + b[...]


def _post(parts, se, skip, aw, ab, g, b):
    blk = 2000
    return pl.pallas_call(
        _post_body, grid=(N // blk,),
        in_specs=[pl.BlockSpec((NC, blk, OUT), lambda i: (0, i, 0)),
                  pl.BlockSpec((1, OUT), lambda i: (0, 0)),
                  pl.BlockSpec((blk, OUT), lambda i: (i, 0)),
                  pl.BlockSpec((OUT, OUT), lambda i: (0, 0)),
                  pl.BlockSpec((1, OUT), lambda i: (0, 0)),
                  pl.BlockSpec((1, OUT), lambda i: (0, 0)),
                  pl.BlockSpec((1, OUT), lambda i: (0, 0))],
        out_specs=pl.BlockSpec((blk, OUT), lambda i: (i, 0)),
        out_shape=jax.ShapeDtypeStruct((N, OUT), f32),
    )(parts, se, skip, aw, ab, g, b)


def kernel(x_n0, x_n1, he_index_n0, he_index_n1, max_he_id,
           k_W_n0, k_b_n0, q_W_n0, q_b_n0, a_W_n0, a_b_n0,
           skip_W_n0, skip_b_n0, ln_g_n0, ln_b_n0,
           k_W_n1, k_b_n1, q_W_n1, q_b_n1, a_W_n1, a_b_n1,
           skip_W_n1, skip_b_n1, ln_g_n1, ln_b_n1):
    del max_he_id

    ni0, hi0 = he_index_n0[0], he_index_n0[1]
    ni1, hi1 = he_index_n1[0], he_index_n1[1]

    wqs0 = jnp.concatenate([q_W_n0, skip_W_n0], axis=1)
    bqs0 = jnp.concatenate([q_b_n0, skip_b_n0])[None, :]
    wqs1 = jnp.concatenate([q_W_n1, skip_W_n1], axis=1)
    bqs1 = jnp.concatenate([q_b_n1, skip_b_n1])[None, :]

    eye = jnp.eye(HEADS, dtype=f32)
    g = jnp.repeat(eye, DK, axis=0)
    gt = jnp.repeat(eye, DK, axis=1)
    zeros = jnp.zeros((N, OUT), f32)

    hk0, hk1 = _dense_hk(x_n0, x_n1, k_W_n0, k_b_n0[None, :],
                         k_W_n1, k_b_n1[None, :])
    hef_parts = _sc_hef(hk0, hk1, ni0, hi0, ni1, hi1, zeros)
    q0, s0, q1, s1 = _dense_qs(x_n0, x_n1, wqs0, bqs0, wqs1, bqs1)
    hef, mq0, mq1, mh = _combine_norms(hef_parts, q0, q1, g)

    qg0, kg0 = _sc_qk(q0, hef, ni0, hi0)
    qg1, kg1 = _sc_qk(q1, hef, ni1, hi1)
    val0, se0 = _alphaval(qg0, kg0, mq0, mh, g, gt)
    val1, se1 = _alphaval(qg1, kg1, mq1, mh, g, gt)

    parts0 = _sc_nodeout(val0, ni0, zeros)
    parts1 = _sc_nodeout(val1, ni1, zeros)

    out0 = _post(parts0, se0, s0, a_W_n0, a_b_n0[None, :],
                 ln_g_n0[None, :], ln_b_n0[None, :])
    out1 = _post(parts1, se1, s1, a_W_n1, a_b_n1[None, :],
                 ln_g_n1[None, :], ln_b_n1[None, :])
    return (out0, out1)


# speedup vs baseline: 112.9730x; 1.0044x over previous
import functools
import math

import jax
import jax.numpy as jnp
from jax import lax
from jax.experimental import pallas as pl
from jax.experimental.pallas import tpu as pltpu
from jax.experimental.pallas import tpu_sc as plsc

N = 10000
D = 128
OUT = 128
HEADS = 8
DK = OUT // HEADS
E = 160000
NC = 2
NS = 16
NW = NC * NS
CHUNK = 128
NCHUNK = E // CHUNK
SLAB = 640


def _per_sub_rows(sid, copy_fn):
    @pl.when(sid < NS - 1)
    def _():
        copy_fn(pl.multiple_of(sid * SLAB, SLAB), SLAB)

    @pl.when(sid == NS - 1)
    def _():
        copy_fn((NS - 1) * SLAB, N - (NS - 1) * SLAB)
@functools.cache
def _mesh():
    return plsc.VectorSubcoreMesh(core_axis_name="c", subcore_axis_name="s",
                                  num_cores=NC, num_subcores=NS)

f32 = jnp.float32


def _dense_hk_body(x0, x1, w0, b0, w1, b1, hk0, hk1):
    hk0[...] = jnp.dot(x0[...], w0[...], preferred_element_type=f32) + b0[...]
    hk1[...] = jnp.dot(x1[...], w1[...], preferred_element_type=f32) + b1[...]


def _dense_hk(x0, x1, w0, b0, w1, b1):
    blk = 2000
    io = pl.BlockSpec((blk, D), lambda i: (i, 0))
    w = pl.BlockSpec((D, OUT), lambda i: (0, 0))
    b = pl.BlockSpec((1, OUT), lambda i: (0, 0))
    o = pl.BlockSpec((blk, OUT), lambda i: (i, 0))
    sh = jax.ShapeDtypeStruct((N, OUT), f32)
    return pl.pallas_call(
        _dense_hk_body, grid=(N // blk,),
        in_specs=[io, io, w, b, w, b],
        out_specs=[o, o],
        out_shape=[sh, sh],
    )(x0, x1, w0, b0, w1, b1)


def _dense_qs_body(x0, x1, w0, b0, w1, b1, g, q0, s0, q1, s1, mq0, mq1):
    i = pl.program_id(0)
    r0 = jnp.dot(x0[...], w0[...], preferred_element_type=f32) + b0[...]
    qa = r0[:, 0:OUT]
    q0[...] = qa
    s0[...] = r0[:, OUT:2 * OUT]
    r1 = jnp.dot(x1[...], w1[...], preferred_element_type=f32) + b1[...]
    qb = r1[:, 0:OUT]
    q1[...] = qb
    s1[...] = r1[:, OUT:2 * OUT]
    neg = jnp.full((1, HEADS), -jnp.inf, f32)
    for ref, x in ((mq0, qa), (mq1, qb)):
        n = jnp.sqrt(jnp.dot(x * x, g[...], preferred_element_type=f32))
        bm = jnp.max(n, axis=0, keepdims=True)
        ref[...] = jnp.maximum(jnp.where(i == 0, neg, ref[...]), bm)


def _dense_qs(x0, x1, w0, b0, w1, b1, g):
    blk = 2000
    io = pl.BlockSpec((blk, D), lambda i: (i, 0))
    w = pl.BlockSpec((D, 2 * OUT), lambda i: (0, 0))
    b = pl.BlockSpec((1, 2 * OUT), lambda i: (0, 0))
    o = pl.BlockSpec((blk, OUT), lambda i: (i, 0))
    shf = jax.ShapeDtypeStruct((N, OUT), f32)
    m = pl.BlockSpec((1, HEADS), lambda i: (0, 0))
    msh = jax.ShapeDtypeStruct((1, HEADS), f32)
    return pl.pallas_call(
        _dense_qs_body, grid=(N // blk,),
        in_specs=[io, io, w, b, w, b,
                  pl.BlockSpec((OUT, HEADS), lambda i: (0, 0))],
        out_specs=[o] * 4 + [m, m],
        out_shape=[shf, shf, shf, shf, msh, msh],
    )(x0, x1, w0, b0, w1, b1, g)


EPW = 4992
NFULL = EPW // CHUNK
TAIL = NFULL * CHUNK * NW


def _sc_hef_body(hk0, hk1, ni0, hi0, ni1, hi1, zeros, out,
                 nvf, nve, hv0, hv1, rows0, rows1, acc,
                 sg0, sg1, sh0, sh1, ss):
    cid = lax.axis_index("c")
    sid = lax.axis_index("s")
    wid = sid * NC + cid
    ebase = pl.multiple_of(wid * EPW, CHUNK)
    nch = jnp.where(wid < 2, NFULL + 1, NFULL)
    hv = (hv0, hv1)
    rows = (rows0, rows1)
    sg = (sg0, sg1)
    sh = (sh0, sh1)
    _per_sub_rows(sid, lambda off, sz: pltpu.sync_copy(
        zeros.at[pl.ds(off, sz)], acc.at[pl.ds(off, sz)]))
    plsc.subcore_barrier()
    for hk, ni, hi in ((hk0, ni0, hi0), (hk1, ni1, hi1)):
        pltpu.async_copy(ni.at[pl.ds(ebase, EPW)], nvf, ss).wait()

        @pl.when(wid < 2)
        def _():
            pltpu.sync_copy(ni.at[pl.ds(pl.multiple_of(TAIL + wid * CHUNK,
                                                       CHUNK), CHUNK)], nve)

        def issue(c, p):
            @pl.when(c < NFULL)
            def _():
                b = pl.multiple_of(c * CHUNK, CHUNK)
                pltpu.async_copy(hi.at[pl.ds(pl.multiple_of(ebase + b, CHUNK),
                                             CHUNK)], hv[p], sh[p])
                pltpu.async_copy(hk.at[nvf.at[pl.ds(b, CHUNK)]], rows[p], sg[p])

            @pl.when(c == NFULL)
            def _():
                off = pl.multiple_of(TAIL + wid * CHUNK, CHUNK)
                pltpu.async_copy(hi.at[pl.ds(off, CHUNK)], hv[p], sh[p])
                pltpu.async_copy(hk.at[nve], rows[p], sg[p])

        def finish(c, p):
            @pl.when(c < nch)
            def _():
                pltpu.make_async_copy(hi.at[pl.ds(0, CHUNK)], hv[p], sh[p]).wait()
                pltpu.make_async_copy(hk.at[nve], rows[p], sg[p]).wait()
                pltpu.sync_copy(rows[p], acc.at[hv[p]], add=True)

        for p in range(2):
            issue(jnp.int32(p), p)

        @pl.loop(0, (NFULL + 2) // 2)
        def _(j):
            for p in range(2):
                c = 2 * j + p
                finish(c, p)

                @pl.when(c + 2 < nch)
                def _():
                    issue(c + 2, p)
    plsc.subcore_barrier()
    _per_sub_rows(sid, lambda off, sz: pltpu.sync_copy(
        acc.at[pl.ds(off, sz)], out.at[cid, pl.ds(off, sz)]))


def _sc_hef(hk0, hk1, ni0, hi0, ni1, hi1, zeros):
    return pl.kernel(
        _sc_hef_body,
        out_type=jax.ShapeDtypeStruct((NC, N, OUT), f32),
        mesh=_mesh(),
        scratch_types=[
            pltpu.VMEM((EPW,), jnp.int32),
            pltpu.VMEM((CHUNK,), jnp.int32),
            pltpu.VMEM((CHUNK,), jnp.int32),
            pltpu.VMEM((CHUNK,), jnp.int32),
            pltpu.VMEM((CHUNK, OUT), f32),
            pltpu.VMEM((CHUNK, OUT), f32),
            pltpu.VMEM_SHARED((N, OUT), f32),
        ] + [pltpu.SemaphoreType.DMA] * 5,
    )(hk0, hk1, ni0, hi0, ni1, hi1, zeros)


def _combine_norms_body(p, g, hef, mh):
    i = pl.program_id(0)
    h = p[0] + p[1]
    hef[...] = h
    n = jnp.sqrt(jnp.dot(h * h, g[...], preferred_element_type=f32))
    bm = jnp.max(n, axis=0, keepdims=True)
    neg = jnp.full((1, HEADS), -jnp.inf, f32)
    mh[...] = jnp.maximum(jnp.where(i == 0, neg, mh[...]), bm)


def _combine_norms(parts, g):
    blk = 2000
    m = pl.BlockSpec((1, HEADS), lambda i: (0, 0))
    msh = jax.ShapeDtypeStruct((1, HEADS), f32)
    return pl.pallas_call(
        _combine_norms_body, grid=(N // blk,),
        in_specs=[pl.BlockSpec((NC, blk, OUT), lambda i: (0, i, 0)),
                  pl.BlockSpec((OUT, HEADS), lambda i: (0, 0))],
        out_specs=[pl.BlockSpec((blk, OUT), lambda i: (i, 0)), m],
        out_shape=[jax.ShapeDtypeStruct((N, OUT), f32), msh],
    )(parts, g)


def _sc_qk_body(q, hef, ni, hi,
                qg, kg,
                nvf, hvf, nve, hve,
                qr0, qr1, qr2, kr0, kr1, kr2,
                sq0, sq1, sq2, sk0, sk1, sk2,
                wq0, wq1, wq2, wk0, wk1, wk2, ss):
    cid = lax.axis_index("c")
    sid = lax.axis_index("s")
    wid = sid * NC + cid
    ebase = pl.multiple_of(wid * EPW, CHUNK)
    nch = jnp.where(wid < 2, NFULL + 1, NFULL)
    qr = (qr0, qr1, qr2)
    kr = (kr0, kr1, kr2)
    sq = (sq0, sq1, sq2)
    sk = (sk0, sk1, sk2)
    wq = (wq0, wq1, wq2)
    wk = (wk0, wk1, wk2)
    cpa = pltpu.async_copy(ni.at[pl.ds(ebase, EPW)], nvf, ss)
    cpb = pltpu.async_copy(hi.at[pl.ds(ebase, EPW)], hvf, ss)
    cpa.wait()
    cpb.wait()

    @pl.when(wid < 2)
    def _():
        toff = pl.multiple_of(TAIL + wid * CHUNK, CHUNK)
        pltpu.sync_copy(ni.at[pl.ds(toff, CHUNK)], nve)
        pltpu.sync_copy(hi.at[pl.ds(toff, CHUNK)], hve)

    def pre(c, p):
        @pl.when(c < nch)
        def _():
            @pl.when(c >= 3)
            def _():
                pltpu.make_async_copy(qr[p], qg.at[pl.ds(0, CHUNK)], wq[p]).wait()
                pltpu.make_async_copy(kr[p], kg.at[pl.ds(0, CHUNK)], wk[p]).wait()

            @pl.when(c < NFULL)
            def _():
                b = pl.multiple_of(c * CHUNK, CHUNK)
                pltpu.async_copy(q.at[nvf.at[pl.ds(b, CHUNK)]], qr[p], sq[p])
                pltpu.async_copy(hef.at[hvf.at[pl.ds(b, CHUNK)]], kr[p], sk[p])

            @pl.when(c == NFULL)
            def _():
                pltpu.async_copy(q.at[nve], qr[p], sq[p])
                pltpu.async_copy(hef.at[hve], kr[p], sk[p])

    def proc(c, p):
        @pl.when(c < nch)
        def _():
            pltpu.make_async_copy(q.at[nve], qr[p], sq[p]).wait()
            pltpu.make_async_copy(hef.at[hve], kr[p], sk[p]).wait()
            woff = pl.multiple_of(
                jnp.where(c < NFULL, ebase + c * CHUNK,
                          TAIL + wid * CHUNK), CHUNK)
            pltpu.async_copy(qr[p], qg.at[pl.ds(woff, CHUNK)], wq[p])
            pltpu.async_copy(kr[p], kg.at[pl.ds(woff, CHUNK)], wk[p])

    for p in range(3):
        pre(jnp.int32(p), p)

    @pl.loop(0, (NFULL + 3) // 3)
    def _(j):
        for p in range(3):
            c = 3 * j + p
            proc(c, p)
            pre(c + 3, p)

    for p in range(3):
        pltpu.make_async_copy(qr[p], qg.at[pl.ds(0, CHUNK)], wq[p]).wait()
        pltpu.make_async_copy(kr[p], kg.at[pl.ds(0, CHUNK)], wk[p]).wait()


def _sc_qk(q, hef, ni, hi):
    sh = jax.ShapeDtypeStruct((E, OUT), f32)
    return pl.kernel(
        _sc_qk_body,
        out_type=(sh, sh),
        mesh=_mesh(),
        scratch_types=[
            pltpu.VMEM((EPW,), jnp.int32),
            pltpu.VMEM((EPW,), jnp.int32),
            pltpu.VMEM((CHUNK,), jnp.int32),
            pltpu.VMEM((CHUNK,), jnp.int32),
            pltpu.VMEM((CHUNK, OUT), f32),
            pltpu.VMEM((CHUNK, OUT), f32),
            pltpu.VMEM((CHUNK, OUT), f32),
            pltpu.VMEM((CHUNK, OUT), f32),
            pltpu.VMEM((CHUNK, OUT), f32),
            pltpu.VMEM((CHUNK, OUT), f32),
        ] + [pltpu.SemaphoreType.DMA] * 13,
    )(q, hef, ni, hi)


def _alphaval_body(qg, kg, mq, mh, g, gt, val, se):
    i = pl.program_id(0)
    kgv = kg[...]
    a = jnp.dot(qg[...] * kgv, g[...],
                preferred_element_type=f32) * (1.0 / math.sqrt(DK))
    bound = mq[...] * mh[...] * (1.0 / math.sqrt(DK)) + 1.0
    w = jnp.exp(a - bound)
    w128 = jnp.dot(w, gt[...], preferred_element_type=f32)
    val[...] = kgv * w128
    prev = jnp.where(i == 0, jnp.zeros((1, OUT), f32), se[...])
    se[...] = prev + jnp.sum(w128, axis=0, keepdims=True)


def _alphaval(qg, kg, mq, mh, g, gt):
    blk = 2000
    io = pl.BlockSpec((blk, OUT), lambda i: (i, 0))
    m = pl.BlockSpec((1, HEADS), lambda i: (0, 0))
    return pl.pallas_call(
        _alphaval_body, grid=(E // blk,),
        in_specs=[io, io, m, m,
                  pl.BlockSpec((OUT, HEADS), lambda i: (0, 0)),
                  pl.BlockSpec((HEADS, OUT), lambda i: (0, 0))],
        out_specs=[pl.BlockSpec((blk, OUT), lambda i: (i, 0)),
                   pl.BlockSpec((1, OUT), lambda i: (0, 0))],
        out_shape=[jax.ShapeDtypeStruct((E, OUT), f32),
                   jax.ShapeDtypeStruct((1, OUT), f32)],
    )(qg, kg, mq, mh, g, gt)


def _sc_nodeout_body(val, ni, zeros, out,
                     nv0, nv1, nv2, rv0, rv1, rv2, acc,
                     sn0, sn1, sn2, sv0, sv1, sv2):
    cid = lax.axis_index("c")
    sid = lax.axis_index("s")
    wid = sid * NC + cid
    ebase = pl.multiple_of(wid * EPW, CHUNK)
    nch = jnp.where(wid < 2, NFULL + 1, NFULL)
    nv = (nv0, nv1, nv2)
    rv = (rv0, rv1, rv2)
    sn = (sn0, sn1, sn2)
    sv = (sv0, sv1, sv2)
    _per_sub_rows(sid, lambda off, sz: pltpu.sync_copy(
        zeros.at[pl.ds(off, sz)], acc.at[pl.ds(off, sz)]))
    plsc.subcore_barrier()

    def issue(c, p):
        off = pl.multiple_of(
            jnp.where(c < NFULL, ebase + c * CHUNK,
                      TAIL + wid * CHUNK), CHUNK)
        pltpu.async_copy(ni.at[pl.ds(off, CHUNK)], nv[p], sn[p])
        pltpu.async_copy(val.at[pl.ds(off, CHUNK)], rv[p], sv[p])

    def finish(c, p):
        @pl.when(c < nch)
        def _():
            pltpu.make_async_copy(ni.at[pl.ds(0, CHUNK)], nv[p], sn[p]).wait()
            pltpu.make_async_copy(val.at[pl.ds(0, CHUNK)], rv[p], sv[p]).wait()
            pltpu.sync_copy(rv[p], acc.at[nv[p]], add=True)

    for p in range(3):
        issue(jnp.int32(p), p)

    @pl.loop(0, (NFULL + 3) // 3)
    def _(j):
        for p in range(3):
            c = 3 * j + p
            finish(c, p)

            @pl.when(c + 3 < nch)
            def _():
                issue(c + 3, p)
    plsc.subcore_barrier()
    _per_sub_rows(sid, lambda off, sz: pltpu.sync_copy(
        acc.at[pl.ds(off, sz)], out.at[cid, pl.ds(off, sz)]))


def _sc_nodeout(val, ni, zeros):
    return pl.kernel(
        _sc_nodeout_body,
        out_type=jax.ShapeDtypeStruct((NC, N, OUT), f32),
        mesh=_mesh(),
        scratch_types=[
            pltpu.VMEM((CHUNK,), jnp.int32),
            pltpu.VMEM((CHUNK,), jnp.int32),
            pltpu.VMEM((CHUNK,), jnp.int32),
            pltpu.VMEM((CHUNK, OUT), f32),
            pltpu.VMEM((CHUNK, OUT), f32),
            pltpu.VMEM((CHUNK, OUT), f32),
            pltpu.VMEM_SHARED((N, OUT), f32),
        ] + [pltpu.SemaphoreType.DMA] * 6,
    )(val, ni, zeros)


def _post_body(parts, se, skip, aw, ab, g, b, o):
    rec = 1.0 / se[...]
    no = (parts[0] + parts[1]) * rec
    merged = jnp.dot(no, aw[...], preferred_element_type=f32) + ab[...]
    y = merged + skip[...]
    mu = jnp.mean(y, axis=-1, keepdims=True)
    var = jnp.mean((y - mu) ** 2, axis=-1, keepdims=True)
    o[...] = (y - mu) * lax.rsqrt(var + 1e-5) * g[...] + b[...]


def _post(parts, se, skip, aw, ab, g, b):
    blk = 2000
    return pl.pallas_call(
        _post_body, grid=(N // blk,),
        in_specs=[pl.BlockSpec((NC, blk, OUT), lambda i: (0, i, 0)),
                  pl.BlockSpec((1, OUT), lambda i: (0, 0)),
                  pl.BlockSpec((blk, OUT), lambda i: (i, 0)),
                  pl.BlockSpec((OUT, OUT), lambda i: (0, 0)),
                  pl.BlockSpec((1, OUT), lambda i: (0, 0)),
                  pl.BlockSpec((1, OUT), lambda i: (0, 0)),
                  pl.BlockSpec((1, OUT), lambda i: (0, 0))],
        out_specs=pl.BlockSpec((blk, OUT), lambda i: (i, 0)),
        out_shape=jax.ShapeDtypeStruct((N, OUT), f32),
    )(parts, se, skip, aw, ab, g, b)


def kernel(x_n0, x_n1, he_index_n0, he_index_n1, max_he_id,
           k_W_n0, k_b_n0, q_W_n0, q_b_n0, a_W_n0, a_b_n0,
           skip_W_n0, skip_b_n0, ln_g_n0, ln_b_n0,
           k_W_n1, k_b_n1, q_W_n1, q_b_n1, a_W_n1, a_b_n1,
           skip_W_n1, skip_b_n1, ln_g_n1, ln_b_n1):
    del max_he_id

    ni0, hi0 = he_index_n0[0], he_index_n0[1]
    ni1, hi1 = he_index_n1[0], he_index_n1[1]

    wqs0 = jnp.concatenate([q_W_n0, skip_W_n0], axis=1)
    bqs0 = jnp.concatenate([q_b_n0, skip_b_n0])[None, :]
    wqs1 = jnp.concatenate([q_W_n1, skip_W_n1], axis=1)
    bqs1 = jnp.concatenate([q_b_n1, skip_b_n1])[None, :]

    eye = jnp.eye(HEADS, dtype=f32)
    g = jnp.repeat(eye, DK, axis=0)
    gt = jnp.repeat(eye, DK, axis=1)
    zeros = jnp.zeros((N, OUT), f32)

    hk0, hk1 = _dense_hk(x_n0, x_n1, k_W_n0, k_b_n0[None, :],
                         k_W_n1, k_b_n1[None, :])
    hef_parts = _sc_hef(hk0, hk1, ni0, hi0, ni1, hi1, zeros)
    q0, s0, q1, s1, mq0, mq1 = _dense_qs(x_n0, x_n1, wqs0, bqs0,
                                         wqs1, bqs1, g)
    hef, mh = _combine_norms(hef_parts, g)

    qg0, kg0 = _sc_qk(q0, hef, ni0, hi0)
    qg1, kg1 = _sc_qk(q1, hef, ni1, hi1)
    val0, se0 = _alphaval(qg0, kg0, mq0, mh, g, gt)
    val1, se1 = _alphaval(qg1, kg1, mq1, mh, g, gt)

    parts0 = _sc_nodeout(val0, ni0, zeros)
    parts1 = _sc_nodeout(val1, ni1, zeros)

    out0 = _post(parts0, se0, s0, a_W_n0, a_b_n0[None, :],
                 ln_g_n0[None, :], ln_b_n0[None, :])
    out1 = _post(parts1, se1, s1, a_W_n1, a_b_n1[None, :],
                 ln_g_n1[None, :], ln_b_n1[None, :])
    return (out0, out1)
